# Initial kernel scaffold; baseline (speedup 1.0000x reference)
import functools

import jax
import jax.numpy as jnp
from jax import lax
from jax.experimental import pallas as pl
from jax.experimental.pallas import tpu as pltpu
from jax.experimental.pallas import tpu_sc as plsc

_N = 10000
_E = 320000
_NH = 128
_NCLASS = 64
_PACK = 144
_K = 128
_NCH = _E // _K
_NC = 2
_NS = 16
_NW = _NC * _NS
_RPT = _N // _NS


def _dense_stage(h_src_arr, W, aw, ab, R=1000):
    def body(x_ref, w_ref, aw_ref, ab_ref, hp_ref, s1_ref):
        h = jnp.dot(x_ref[...], w_ref[...], preferred_element_type=jnp.float32)
        aw_v = aw_ref[...]
        s_dst = jnp.dot(h, aw_v[128:256], preferred_element_type=jnp.float32)
        s_src = jnp.dot(h, aw_v[0:128], preferred_element_type=jnp.float32) + ab_ref[0]
        hp_ref[...] = jnp.concatenate(
            [h, s_dst, jnp.zeros((R, _PACK - _NH - 1), jnp.float32)], axis=1)
        s1_ref[...] = s_src

    return pl.pallas_call(
        body,
        grid=(_N // R,),
        in_specs=[
            pl.BlockSpec((R, _NH), lambda i: (i, 0)),
            pl.BlockSpec((_NH, _NH), lambda i: (0, 0)),
            pl.BlockSpec((2 * _NH, 1), lambda i: (0, 0)),
            pl.BlockSpec(memory_space=pltpu.SMEM),
        ],
        out_specs=[
            pl.BlockSpec((R, _PACK), lambda i: (i, 0)),
            pl.BlockSpec((R, 1), lambda i: (i, 0)),
        ],
        out_shape=[
            jax.ShapeDtypeStruct((_N, _PACK), jnp.float32),
            jax.ShapeDtypeStruct((_N, 1), jnp.float32),
        ],
    )(h_src_arr, W, aw, ab)


def _mid_stage(acc, W, aw, ab, R=1000):
    def body(acc_ref, w_ref, aw_ref, ab_ref, hp_ref, s1_ref):
        sacc = acc_ref[0] + acc_ref[1]
        den = sacc[:, 128:129]
        den = jnp.where(den > 0.0, den, 1.0)
        h_in = jnp.maximum(sacc[:, 0:_NH] / den, 0.0)
        h = jnp.dot(h_in, w_ref[...], preferred_element_type=jnp.float32)
        aw_v = aw_ref[...]
        s_dst = jnp.dot(h, aw_v[128:256], preferred_element_type=jnp.float32)
        s_src = jnp.dot(h, aw_v[0:128], preferred_element_type=jnp.float32) + ab_ref[0]
        hp_ref[...] = jnp.concatenate(
            [h, s_dst, jnp.zeros((R, _PACK - _NH - 1), jnp.float32)], axis=1)
        s1_ref[...] = s_src

    return pl.pallas_call(
        body,
        grid=(_N // R,),
        in_specs=[
            pl.BlockSpec((2, R, _PACK), lambda i: (0, i, 0)),
            pl.BlockSpec((_NH, _NH), lambda i: (0, 0)),
            pl.BlockSpec((2 * _NH, 1), lambda i: (0, 0)),
            pl.BlockSpec(memory_space=pltpu.SMEM),
        ],
        out_specs=[
            pl.BlockSpec((R, _PACK), lambda i: (i, 0)),
            pl.BlockSpec((R, 1), lambda i: (i, 0)),
        ],
        out_shape=[
            jax.ShapeDtypeStruct((_N, _PACK), jnp.float32),
            jax.ShapeDtypeStruct((_N, 1), jnp.float32),
        ],
    )(acc, W, aw, ab)


def _final_stage(acc, fc_w, fc_b, R=1000):
    def body(acc_ref, w_ref, b_ref, out_ref, den_ref):
        sacc = acc_ref[0] + acc_ref[1]
        den = sacc[:, 128:129]
        den_s = jnp.where(den > 0.0, den, 1.0)
        h = jnp.maximum(sacc[:, 0:_NH] / den_s, 0.0)
        logits = jnp.dot(h, w_ref[...], preferred_element_type=jnp.float32) + b_ref[...]
        m = jnp.max(logits, axis=1, keepdims=True)
        lse = m + jnp.log(jnp.sum(jnp.exp(logits - m), axis=1, keepdims=True))
        out_ref[...] = logits - lse
        den_ref[...] = den_s

    return pl.pallas_call(
        body,
        grid=(_N // R,),
        in_specs=[
            pl.BlockSpec((2, R, _PACK), lambda i: (0, i, 0)),
            pl.BlockSpec((_NH, _NCLASS), lambda i: (0, 0)),
            pl.BlockSpec((1, _NCLASS), lambda i: (0, 0)),
        ],
        out_specs=[
            pl.BlockSpec((R, _NCLASS), lambda i: (i, 0)),
            pl.BlockSpec((R, 1), lambda i: (i, 0)),
        ],
        out_shape=[
            jax.ShapeDtypeStruct((_N, _NCLASS), jnp.float32),
            jax.ShapeDtypeStruct((_N, 1), jnp.float32),
        ],
    )(acc, fc_w, fc_b.reshape(1, _NCLASS))


_MESH = plsc.VectorSubcoreMesh(core_axis_name="c", subcore_axis_name="s")


def _make_agg(want_e):
    out_type = [jax.ShapeDtypeStruct((_NC, _N, _PACK), jnp.float32)]
    if want_e:
        out_type.append(jax.ShapeDtypeStruct((_NCH, _K), jnp.float32))

    def body(hp, s1, srcH, dstH, *rest):
        if want_e:
            accO, eO, acc_sh, srcb, dstb, ssrcb, ebuf, rowbuf, sem = rest
        else:
            accO, acc_sh, srcb, dstb, ssrcb, ebuf, rowbuf, sem = rest
        c = lax.axis_index("c")
        s = lax.axis_index("s")
        wid = s * _NC + c

        zv = jnp.zeros((16,), jnp.float32)

        def zrow(i, carry):
            for g in range(_PACK // 16):
                rowbuf[i, pl.ds(g * 16, 16)] = zv
            return carry

        lax.fori_loop(0, _K, zrow, 0)
        base = s * _RPT
        for t in range(_RPT // _K):
            pltpu.sync_copy(rowbuf, acc_sh.at[pl.ds(base + t * _K, _K)])
        rem = _RPT % _K
        pltpu.sync_copy(rowbuf.at[pl.ds(0, rem)],
                        acc_sh.at[pl.ds(base + (_RPT // _K) * _K, rem)])
        plsc.subcore_barrier()

        nch_mine = lax.select(wid < _NCH % _NW,
                              jnp.int32(_NCH // _NW + 1), jnp.int32(_NCH // _NW))

        def chunk(k, carry):
            j = wid + k * _NW
            pltpu.sync_copy(srcH.at[j], srcb)
            pltpu.sync_copy(dstH.at[j], dstb)
            pltpu.async_copy(s1.at[srcb], ssrcb, sem).wait()
            pltpu.async_copy(hp.at[dstb], rowbuf, sem).wait()

            def egrp(g, carry2):
                rid = lax.iota(jnp.int32, 16) + g * 16
                s_dst = plsc.load_gather(
                    rowbuf, [rid, jnp.full((16,), _NH, jnp.int32)])
                z = ssrcb[pl.ds(g * 16, 16)] + s_dst
                z = jnp.maximum(z, 0.05 * z)
                ebuf[pl.ds(g * 16, 16)] = jnp.exp(z)
                return carry2

            lax.fori_loop(0, _K // 16, egrp, 0)

            lane0 = lax.iota(jnp.int32, 16) == 0

            def rscale(i, carry2):
                ev = ebuf[i]
                for g in range(_NH // 16):
                    rowbuf[i, pl.ds(g * 16, 16)] = rowbuf[i, pl.ds(g * 16, 16)] * ev
                rowbuf[i, pl.ds(_NH, 16)] = jnp.where(lane0, ev, 0.0)
                return carry2

            lax.fori_loop(0, _K, rscale, 0)

            pltpu.sync_copy(rowbuf, acc_sh.at[srcb], add=True)
            if want_e:
                pltpu.sync_copy(ebuf, eO.at[j])
            return carry

        lax.fori_loop(0, nch_mine, chunk, 0)
        plsc.subcore_barrier()

        pltpu.sync_copy(acc_sh.at[pl.ds(base, _RPT)],
                        accO.at[c, pl.ds(base, _RPT)])

    return pl.kernel(
        body,
        out_type=out_type,
        mesh=_MESH,
        scratch_types=[
            pltpu.VMEM_SHARED((_N, _PACK), jnp.float32),
            pltpu.VMEM((_K,), jnp.int32),
            pltpu.VMEM((_K,), jnp.int32),
            pltpu.VMEM((_K,), jnp.float32),
            pltpu.VMEM((_K,), jnp.float32),
            pltpu.VMEM((_K, _PACK), jnp.float32),
            pltpu.SemaphoreType.DMA,
        ],
    )


_agg_noe = _make_agg(False)
_agg_e = _make_agg(True)


def _alpha_pass(e2, den, srcA):
    def body(eH, denH, srcH, aO, srcb, ebuf, dbuf, abuf, sem):
        c = lax.axis_index("c")
        s = lax.axis_index("s")
        wid = s * _NC + c
        nch_mine = lax.select(wid < _NCH % _NW,
                              jnp.int32(_NCH // _NW + 1), jnp.int32(_NCH // _NW))

        def chunk(k, carry):
            j = wid + k * _NW
            pltpu.sync_copy(srcH.at[j], srcb)
            pltpu.sync_copy(eH.at[j], ebuf)
            pltpu.async_copy(denH.at[srcb], dbuf, sem).wait()

            def grp(g, carry2):
                sl = pl.ds(g * 16, 16)
                abuf[sl] = ebuf[sl] / dbuf[sl]
                return carry2

            lax.fori_loop(0, _K // 16, grp, 0)
            pltpu.sync_copy(abuf, aO.at[j])
            return carry

        lax.fori_loop(0, nch_mine, chunk, 0)

    return pl.kernel(
        body,
        out_type=jax.ShapeDtypeStruct((_NCH, _K), jnp.float32),
        mesh=_MESH,
        scratch_types=[
            pltpu.VMEM((_K,), jnp.int32),
            pltpu.VMEM((_K,), jnp.float32),
            pltpu.VMEM((_K,), jnp.float32),
            pltpu.VMEM((_K,), jnp.float32),
            pltpu.SemaphoreType.DMA,
        ],
    )(e2, den, srcA)


def kernel(x, edge_index, W1, a1_w, a1_b, W2, a2_w, a2_b, fc_w, fc_b):
    src = edge_index[0].reshape(_NCH, _K)
    dst = edge_index[1].reshape(_NCH, _K)

    hp1, s1 = _dense_stage(x, W1, a1_w, a1_b)
    (acc1,) = _agg_noe(hp1, s1.reshape(_N), src, dst)
    hp2, s2 = _mid_stage(acc1, W2, a2_w, a2_b)
    acc2, e2 = _agg_e(hp2, s2.reshape(_N), src, dst)
    out, den2 = _final_stage(acc2, fc_w, fc_b)
    alpha = _alpha_pass(e2, den2.reshape(_N), src)
    return out, alpha.reshape(_E)


# speedup vs baseline: 8.0814x; 8.0814x over previous
import functools

import jax
import jax.numpy as jnp
from jax import lax
from jax.experimental import pallas as pl
from jax.experimental.pallas import tpu as pltpu
from jax.experimental.pallas import tpu_sc as plsc

_N = 10000
_E = 320000
_NH = 128
_NCLASS = 64
_K = 128
_NCH = _E // _K
_NC = 2
_NS = 16
_NW = _NC * _NS
_NPAD = 10240
_RPT = _NPAD // _NS
_R = 1024


def _dense_tail(h, aw_ref, ab_ref, hp_ref, ss_ref, sd_ref):
    aw_v = aw_ref[...]
    s_src = jnp.dot(h, aw_v[0:128], preferred_element_type=jnp.float32) + ab_ref[0]
    s_dst = jnp.dot(h, aw_v[128:256], preferred_element_type=jnp.float32)
    hp_ref[...] = h
    ss_ref[...] = s_src
    sd_ref[...] = s_dst


_NODE_OUT = [
    jax.ShapeDtypeStruct((_NPAD, _NH), jnp.float32),
    jax.ShapeDtypeStruct((_NPAD, 1), jnp.float32),
    jax.ShapeDtypeStruct((_NPAD, 1), jnp.float32),
]
_NODE_SPECS = [
    pl.BlockSpec((_R, _NH), lambda i: (i, 0)),
    pl.BlockSpec((_R, 1), lambda i: (i, 0)),
    pl.BlockSpec((_R, 1), lambda i: (i, 0)),
]


def _first_stage(x_pad, W, aw, ab):
    def body(x_ref, w_ref, aw_ref, ab_ref, hp_ref, ss_ref, sd_ref):
        h = jnp.dot(x_ref[...], w_ref[...], preferred_element_type=jnp.float32)
        _dense_tail(h, aw_ref, ab_ref, hp_ref, ss_ref, sd_ref)

    return pl.pallas_call(
        body,
        grid=(_NPAD // _R,),
        in_specs=[
            pl.BlockSpec((_R, _NH), lambda i: (i, 0)),
            pl.BlockSpec((_NH, _NH), lambda i: (0, 0)),
            pl.BlockSpec((2 * _NH, 1), lambda i: (0, 0)),
            pl.BlockSpec(memory_space=pltpu.SMEM),
        ],
        out_specs=_NODE_SPECS,
        out_shape=_NODE_OUT,
    )(x_pad, W, aw, ab)


def _mid_stage(acc, den, W, aw, ab):
    def body(acc_ref, den_ref, w_ref, aw_ref, ab_ref, hp_ref, ss_ref, sd_ref):
        sacc = acc_ref[0] + acc_ref[1]
        den_v = (den_ref[0] + den_ref[1]).reshape(_R, 1)
        den_v = jnp.where(den_v > 0.0, den_v, 1.0)
        h_in = jnp.maximum(sacc / den_v, 0.0)
        h = jnp.dot(h_in, w_ref[...], preferred_element_type=jnp.float32)
        _dense_tail(h, aw_ref, ab_ref, hp_ref, ss_ref, sd_ref)

    return pl.pallas_call(
        body,
        grid=(_NPAD // _R,),
        in_specs=[
            pl.BlockSpec((2, _R, _NH), lambda i: (0, i, 0)),
            pl.BlockSpec((2, _R), lambda i: (0, i)),
            pl.BlockSpec((_NH, _NH), lambda i: (0, 0)),
            pl.BlockSpec((2 * _NH, 1), lambda i: (0, 0)),
            pl.BlockSpec(memory_space=pltpu.SMEM),
        ],
        out_specs=_NODE_SPECS,
        out_shape=_NODE_OUT,
    )(acc, den, W, aw, ab)


def _final_stage(acc, den, fc_w, fc_b):
    def body(acc_ref, den_ref, w_ref, b_ref, out_ref, den_o_ref):
        sacc = acc_ref[0] + acc_ref[1]
        den_v = (den_ref[0] + den_ref[1]).reshape(_R, 1)
        den_s = jnp.where(den_v > 0.0, den_v, 1.0)
        h = jnp.maximum(sacc / den_s, 0.0)
        logits = jnp.dot(h, w_ref[...], preferred_element_type=jnp.float32) + b_ref[...]
        m = jnp.max(logits, axis=1, keepdims=True)
        lse = m + jnp.log(jnp.sum(jnp.exp(logits - m), axis=1, keepdims=True))
        out_ref[...] = logits - lse
        den_o_ref[...] = den_s

    return pl.pallas_call(
        body,
        grid=(_NPAD // _R,),
        in_specs=[
            pl.BlockSpec((2, _R, _NH), lambda i: (0, i, 0)),
            pl.BlockSpec((2, _R), lambda i: (0, i)),
            pl.BlockSpec((_NH, _NCLASS), lambda i: (0, 0)),
            pl.BlockSpec((1, _NCLASS), lambda i: (0, 0)),
        ],
        out_specs=[
            pl.BlockSpec((_R, _NCLASS), lambda i: (i, 0)),
            pl.BlockSpec((_R, 1), lambda i: (i, 0)),
        ],
        out_shape=[
            jax.ShapeDtypeStruct((_NPAD, _NCLASS), jnp.float32),
            jax.ShapeDtypeStruct((_NPAD, 1), jnp.float32),
        ],
    )(acc, den, fc_w, fc_b.reshape(1, _NCLASS))


@functools.cache
def _get_mesh():
    return plsc.VectorSubcoreMesh(core_axis_name="c", subcore_axis_name="s",
                                  num_cores=_NC, num_subcores=_NS)


@functools.cache
def _make_agg(want_e):
    out_type = [
        jax.ShapeDtypeStruct((_NC, _NPAD, _NH), jnp.float32),
        jax.ShapeDtypeStruct((_NC, _NPAD), jnp.float32),
    ]
    if want_e:
        out_type.append(jax.ShapeDtypeStruct((_E,), jnp.float32))

    def body(hT, ssT, sdT, srcH, dstH, *rest):
        if want_e:
            accO, denO, eO, acc_sh, den_sh, srcb, dstb, ssrcb, sdstb, ebuf, rowbuf, sem = rest
        else:
            accO, denO, acc_sh, den_sh, srcb, dstb, ssrcb, sdstb, ebuf, rowbuf, sem = rest
        c = lax.axis_index("c")
        s = lax.axis_index("s")
        wid = s * _NC + c

        zv = jnp.zeros((16,), jnp.float32)

        def zrow(i, carry):
            for g in range(_NH // 16):
                rowbuf[i, pl.ds(g * 16, 16)] = zv
            return carry

        lax.fori_loop(0, _K, zrow, 0)

        def zden(t, carry):
            ebuf[pl.ds(t * 16, 16)] = zv
            return carry

        lax.fori_loop(0, _K // 16, zden, 0)

        base = s * _RPT
        for t in range(_RPT // _K):
            pltpu.sync_copy(rowbuf, acc_sh.at[pl.ds(base + t * _K, _K)])
            pltpu.sync_copy(ebuf.at[pl.ds(0, _K)],
                            den_sh.at[pl.ds(base + t * _K, _K)])
        plsc.subcore_barrier()

        nch_mine = lax.select(wid < _NCH % _NW,
                              jnp.int32(_NCH // _NW + 1), jnp.int32(_NCH // _NW))

        def chunk(k, carry):
            j = wid + k * _NW
            pltpu.sync_copy(srcH.at[pl.ds(j * _K, _K)], srcb)
            pltpu.sync_copy(dstH.at[pl.ds(j * _K, _K)], dstb)
            pltpu.async_copy(ssT.at[srcb], ssrcb, sem).wait()
            pltpu.async_copy(sdT.at[dstb], sdstb, sem).wait()
            pltpu.async_copy(hT.at[dstb], rowbuf, sem).wait()

            def egrp(g, carry2):
                sl = pl.ds(g * 16, 16)
                z = ssrcb[sl] + sdstb[sl]
                z = jnp.maximum(z, 0.05 * z)
                ebuf[sl] = jnp.exp(z)
                return carry2

            lax.fori_loop(0, _K // 16, egrp, 0)

            def rscale(i, carry2):
                ev = ebuf[pl.ds(i, 16)][0]
                for g in range(_NH // 16):
                    rowbuf[i, pl.ds(g * 16, 16)] = rowbuf[i, pl.ds(g * 16, 16)] * ev
                return carry2

            lax.fori_loop(0, _K, rscale, 0)

            pltpu.sync_copy(rowbuf, acc_sh.at[srcb], add=True)
            pltpu.sync_copy(ebuf.at[pl.ds(0, _K)], den_sh.at[srcb], add=True)
            if want_e:
                pltpu.sync_copy(ebuf.at[pl.ds(0, _K)], eO.at[pl.ds(j * _K, _K)])
            return carry

        lax.fori_loop(0, nch_mine, chunk, 0)
        plsc.subcore_barrier()

        pltpu.sync_copy(acc_sh.at[pl.ds(base, _RPT)],
                        accO.at[c, pl.ds(base, _RPT)])
        pltpu.sync_copy(den_sh.at[pl.ds(base, _RPT)],
                        denO.at[c, pl.ds(base, _RPT)])

    return pl.kernel(
        body,
        out_type=out_type,
        mesh=_get_mesh(),
        scratch_types=[
            pltpu.VMEM_SHARED((_NPAD, _NH), jnp.float32),
            pltpu.VMEM_SHARED((_NPAD,), jnp.float32),
            pltpu.VMEM((_K,), jnp.int32),
            pltpu.VMEM((_K,), jnp.int32),
            pltpu.VMEM((_K,), jnp.float32),
            pltpu.VMEM((_K,), jnp.float32),
            pltpu.VMEM((_K + 16,), jnp.float32),
            pltpu.VMEM((_K, _NH), jnp.float32),
            pltpu.SemaphoreType.DMA,
        ],
    )


def _alpha_pass(e2, den, srcH):
    def body(eH, denH, srcH_, aO, srcb, ebuf, dbuf, abuf, sem):
        c = lax.axis_index("c")
        s = lax.axis_index("s")
        wid = s * _NC + c
        nch_mine = lax.select(wid < _NCH % _NW,
                              jnp.int32(_NCH // _NW + 1), jnp.int32(_NCH // _NW))

        def chunk(k, carry):
            j = wid + k * _NW
            pltpu.sync_copy(srcH_.at[pl.ds(j * _K, _K)], srcb)
            pltpu.sync_copy(eH.at[pl.ds(j * _K, _K)], ebuf)
            pltpu.async_copy(denH.at[srcb], dbuf, sem).wait()

            def grp(g, carry2):
                sl = pl.ds(g * 16, 16)
                abuf[sl] = ebuf[sl] / dbuf[sl]
                return carry2

            lax.fori_loop(0, _K // 16, grp, 0)
            pltpu.sync_copy(abuf, aO.at[pl.ds(j * _K, _K)])
            return carry

        lax.fori_loop(0, nch_mine, chunk, 0)

    return pl.kernel(
        body,
        out_type=jax.ShapeDtypeStruct((_E,), jnp.float32),
        mesh=_get_mesh(),
        scratch_types=[
            pltpu.VMEM((_K,), jnp.int32),
            pltpu.VMEM((_K,), jnp.float32),
            pltpu.VMEM((_K,), jnp.float32),
            pltpu.VMEM((_K,), jnp.float32),
            pltpu.SemaphoreType.DMA,
        ],
    )(e2, den, srcH)


def kernel(x, edge_index, W1, a1_w, a1_b, W2, a2_w, a2_b, fc_w, fc_b):
    src = edge_index[0]
    dst = edge_index[1]
    x_pad = jnp.pad(x, ((0, _NPAD - _N), (0, 0)))

    h1, ss1, sd1 = _first_stage(x_pad, W1, a1_w, a1_b)
    acc1, den1 = _make_agg(False)(
        h1, ss1.reshape(_NPAD), sd1.reshape(_NPAD), src, dst)
    h2, ss2, sd2 = _mid_stage(acc1, den1, W2, a2_w, a2_b)
    acc2, den2, e2 = _make_agg(True)(
        h2, ss2.reshape(_NPAD), sd2.reshape(_NPAD), src, dst)
    out, den2s = _final_stage(acc2, den2, fc_w, fc_b)
    alpha = _alpha_pass(e2, den2s.reshape(_NPAD), src)
    return out[:_N], alpha


# speedup vs baseline: 14.7166x; 1.8211x over previous
import functools

import jax
import jax.numpy as jnp
from jax import lax
from jax.experimental import pallas as pl
from jax.experimental.pallas import tpu as pltpu
from jax.experimental.pallas import tpu_sc as plsc

_N = 10000
_E = 320000
_NH = 128
_NCLASS = 64
_K = 128
_NCH = _E // _K
_NC = 2
_NS = 16
_NW = _NC * _NS
_NPAD = 10240
_RPT = _NPAD // _NS
_R = 1024


def _dense_tail(h, aw_ref, ab_ref, hp_ref, ss_ref, sd_ref):
    aw_v = aw_ref[...]
    s_src = jnp.dot(h, aw_v[0:128], preferred_element_type=jnp.float32) + ab_ref[0]
    s_dst = jnp.dot(h, aw_v[128:256], preferred_element_type=jnp.float32)
    hp_ref[...] = h
    ss_ref[...] = s_src
    sd_ref[...] = s_dst


_NODE_OUT = [
    jax.ShapeDtypeStruct((_NPAD, _NH), jnp.float32),
    jax.ShapeDtypeStruct((_NPAD, 1), jnp.float32),
    jax.ShapeDtypeStruct((_NPAD, 1), jnp.float32),
]
_NODE_SPECS = [
    pl.BlockSpec((_R, _NH), lambda i: (i, 0)),
    pl.BlockSpec((_R, 1), lambda i: (i, 0)),
    pl.BlockSpec((_R, 1), lambda i: (i, 0)),
]


def _first_stage(x_pad, W, aw, ab):
    def body(x_ref, w_ref, aw_ref, ab_ref, hp_ref, ss_ref, sd_ref):
        h = jnp.dot(x_ref[...], w_ref[...], preferred_element_type=jnp.float32)
        _dense_tail(h, aw_ref, ab_ref, hp_ref, ss_ref, sd_ref)

    return pl.pallas_call(
        body,
        grid=(_NPAD // _R,),
        in_specs=[
            pl.BlockSpec((_R, _NH), lambda i: (i, 0)),
            pl.BlockSpec((_NH, _NH), lambda i: (0, 0)),
            pl.BlockSpec((2 * _NH, 1), lambda i: (0, 0)),
            pl.BlockSpec(memory_space=pltpu.SMEM),
        ],
        out_specs=_NODE_SPECS,
        out_shape=_NODE_OUT,
    )(x_pad, W, aw, ab)


def _mid_stage(acc, den, W, aw, ab):
    def body(acc_ref, den_ref, w_ref, aw_ref, ab_ref, hp_ref, ss_ref, sd_ref):
        sacc = acc_ref[0] + acc_ref[1]
        den_v = (den_ref[0] + den_ref[1]).reshape(_R, 1)
        den_v = jnp.where(den_v > 0.0, den_v, 1.0)
        h_in = jnp.maximum(sacc / den_v, 0.0)
        h = jnp.dot(h_in, w_ref[...], preferred_element_type=jnp.float32)
        _dense_tail(h, aw_ref, ab_ref, hp_ref, ss_ref, sd_ref)

    return pl.pallas_call(
        body,
        grid=(_NPAD // _R,),
        in_specs=[
            pl.BlockSpec((2, _R, _NH), lambda i: (0, i, 0)),
            pl.BlockSpec((2, _R), lambda i: (0, i)),
            pl.BlockSpec((_NH, _NH), lambda i: (0, 0)),
            pl.BlockSpec((2 * _NH, 1), lambda i: (0, 0)),
            pl.BlockSpec(memory_space=pltpu.SMEM),
        ],
        out_specs=_NODE_SPECS,
        out_shape=_NODE_OUT,
    )(acc, den, W, aw, ab)


def _final_stage(acc, den, fc_w, fc_b):
    def body(acc_ref, den_ref, w_ref, b_ref, out_ref, den_o_ref):
        sacc = acc_ref[0] + acc_ref[1]
        den_v = (den_ref[0] + den_ref[1]).reshape(_R, 1)
        den_s = jnp.where(den_v > 0.0, den_v, 1.0)
        h = jnp.maximum(sacc / den_s, 0.0)
        logits = jnp.dot(h, w_ref[...], preferred_element_type=jnp.float32) + b_ref[...]
        m = jnp.max(logits, axis=1, keepdims=True)
        lse = m + jnp.log(jnp.sum(jnp.exp(logits - m), axis=1, keepdims=True))
        out_ref[...] = logits - lse
        den_o_ref[...] = den_s

    return pl.pallas_call(
        body,
        grid=(_NPAD // _R,),
        in_specs=[
            pl.BlockSpec((2, _R, _NH), lambda i: (0, i, 0)),
            pl.BlockSpec((2, _R), lambda i: (0, i)),
            pl.BlockSpec((_NH, _NCLASS), lambda i: (0, 0)),
            pl.BlockSpec((1, _NCLASS), lambda i: (0, 0)),
        ],
        out_specs=[
            pl.BlockSpec((_R, _NCLASS), lambda i: (i, 0)),
            pl.BlockSpec((_R, 1), lambda i: (i, 0)),
        ],
        out_shape=[
            jax.ShapeDtypeStruct((_NPAD, _NCLASS), jnp.float32),
            jax.ShapeDtypeStruct((_NPAD, 1), jnp.float32),
        ],
    )(acc, den, fc_w, fc_b.reshape(1, _NCLASS))


@functools.cache
def _get_mesh():
    return plsc.VectorSubcoreMesh(core_axis_name="c", subcore_axis_name="s",
                                  num_cores=_NC, num_subcores=_NS)


@functools.cache
def _make_agg(want_e):
    out_type = [
        jax.ShapeDtypeStruct((_NC, _NPAD, _NH), jnp.float32),
        jax.ShapeDtypeStruct((_NC, _NPAD), jnp.float32),
    ]
    if want_e:
        out_type.append(jax.ShapeDtypeStruct((_E,), jnp.float32))

    def body(hT, ssT, sdT, srcH, dstH, *rest):
        if want_e:
            (accO, denO, eO, acc_sh, den_sh,
             srcbA, dstbA, ssrcbA, sdstbA, ebufA, rowbufA,
             srcbB, dstbB, ssrcbB, sdstbB, ebufB, rowbufB,
             semGA, semGB, semSA, semSB, semEA, semEB) = rest
        else:
            (accO, denO, acc_sh, den_sh,
             srcbA, dstbA, ssrcbA, sdstbA, ebufA, rowbufA,
             srcbB, dstbB, ssrcbB, sdstbB, ebufB, rowbufB,
             semGA, semGB, semSA, semSB, semEA, semEB) = rest
            eO = None
        c = lax.axis_index("c")
        s = lax.axis_index("s")
        wid = s * _NC + c

        zv = jnp.zeros((16,), jnp.float32)

        def zrow(i, carry):
            for g in range(_NH // 16):
                rowbufA[i, pl.ds(g * 16, 16)] = zv
            return carry

        lax.fori_loop(0, _K, zrow, 0)

        def zden(t, carry):
            ebufA[pl.ds(t * 16, 16)] = zv
            return carry

        lax.fori_loop(0, _K // 16, zden, 0)

        base = s * _RPT
        for t in range(_RPT // _K):
            pltpu.sync_copy(rowbufA, acc_sh.at[pl.ds(base + t * _K, _K)])
            pltpu.sync_copy(ebufA.at[pl.ds(0, _K)],
                            den_sh.at[pl.ds(base + t * _K, _K)])
        plsc.subcore_barrier()

        nch_mine = lax.select(wid < _NCH % _NW,
                              jnp.int32(_NCH // _NW + 1), jnp.int32(_NCH // _NW))
        nch2 = (nch_mine + 1) // 2

        def load_idx(j, srcb, dstb):
            pltpu.sync_copy(srcH.at[pl.ds(j * _K, _K)], srcb)
            pltpu.sync_copy(dstH.at[pl.ds(j * _K, _K)], dstb)

        def issue_gathers(srcb, dstb, ssrcb, sdstb, rowbuf, sem):
            pltpu.async_copy(ssT.at[srcb], ssrcb, sem)
            pltpu.async_copy(sdT.at[dstb], sdstb, sem)
            pltpu.async_copy(hT.at[dstb], rowbuf, sem)

        def drain_gathers(ssrcb, sdstb, rowbuf, sem):
            pltpu.make_async_copy(ssT.at[pl.ds(0, _K)], ssrcb, sem).wait()
            pltpu.make_async_copy(sdT.at[pl.ds(0, _K)], sdstb, sem).wait()
            pltpu.make_async_copy(hT.at[pl.ds(0, _K)], rowbuf, sem).wait()

        def compute(ssrcb, sdstb, ebuf, rowbuf):
            def egrp(g, carry2):
                sl = pl.ds(g * 16, 16)
                z = ssrcb[sl] + sdstb[sl]
                z = jnp.maximum(z, 0.05 * z)
                ebuf[sl] = jnp.exp(z)
                return carry2

            lax.fori_loop(0, _K // 16, egrp, 0)

            def rscale(i, carry2):
                ev = ebuf[pl.ds(i, 16)][0]
                for g in range(_NH // 16):
                    rowbuf[i, pl.ds(g * 16, 16)] = rowbuf[i, pl.ds(g * 16, 16)] * ev
                return carry2

            lax.fori_loop(0, _K, rscale, 0)

        def issue_scatters(j, srcb, ebuf, rowbuf, semS, semE):
            pltpu.async_copy(rowbuf, acc_sh.at[srcb], semS, add=True)
            pltpu.async_copy(ebuf.at[pl.ds(0, _K)], den_sh.at[srcb], semS,
                             add=True)
            if want_e:
                pltpu.async_copy(ebuf.at[pl.ds(0, _K)],
                                 eO.at[pl.ds(j * _K, _K)], semE)

        def drain_scatters(ebuf, rowbuf, semS, semE):
            pltpu.make_async_copy(hT.at[pl.ds(0, _K)], rowbuf, semS).wait()
            pltpu.make_async_copy(ssT.at[pl.ds(0, _K)],
                                  ebuf.at[pl.ds(0, _K)], semS).wait()
            if want_e:
                pltpu.make_async_copy(ebuf.at[pl.ds(0, _K)],
                                      eO.at[pl.ds(0, _K)], semE).wait()

        load_idx(wid, srcbA, dstbA)
        issue_gathers(srcbA, dstbA, ssrcbA, sdstbA, rowbufA, semGA)

        def piter(k2, carry):
            k0 = 2 * k2
            j0 = wid + k0 * _NW
            j1 = j0 + _NW
            has1 = k0 + 1 < nch_mine

            @pl.when(has1)
            def _():
                @pl.when(k2 > 0)
                def _():
                    drain_scatters(ebufB, rowbufB, semSB, semEB)
                load_idx(j1, srcbB, dstbB)
                issue_gathers(srcbB, dstbB, ssrcbB, sdstbB, rowbufB, semGB)

            drain_gathers(ssrcbA, sdstbA, rowbufA, semGA)
            compute(ssrcbA, sdstbA, ebufA, rowbufA)
            issue_scatters(j0, srcbA, ebufA, rowbufA, semSA, semEA)

            @pl.when(has1)
            def _():
                drain_gathers(ssrcbB, sdstbB, rowbufB, semGB)
                compute(ssrcbB, sdstbB, ebufB, rowbufB)
                issue_scatters(j1, srcbB, ebufB, rowbufB, semSB, semEB)

            @pl.when(k0 + 2 < nch_mine)
            def _():
                drain_scatters(ebufA, rowbufA, semSA, semEA)
                load_idx(j0 + 2 * _NW, srcbA, dstbA)
                issue_gathers(srcbA, dstbA, ssrcbA, sdstbA, rowbufA, semGA)

            return carry

        lax.fori_loop(0, nch2, piter, 0)
        drain_scatters(ebufA, rowbufA, semSA, semEA)
        drain_scatters(ebufB, rowbufB, semSB, semEB)
        plsc.subcore_barrier()

        pltpu.sync_copy(acc_sh.at[pl.ds(base, _RPT)],
                        accO.at[c, pl.ds(base, _RPT)])
        pltpu.sync_copy(den_sh.at[pl.ds(base, _RPT)],
                        denO.at[c, pl.ds(base, _RPT)])

    return pl.kernel(
        body,
        out_type=out_type,
        mesh=_get_mesh(),
        scratch_types=[
            pltpu.VMEM_SHARED((_NPAD, _NH), jnp.float32),
            pltpu.VMEM_SHARED((_NPAD,), jnp.float32),
        ] + 2 * [
            pltpu.VMEM((_K,), jnp.int32),
            pltpu.VMEM((_K,), jnp.int32),
            pltpu.VMEM((_K,), jnp.float32),
            pltpu.VMEM((_K,), jnp.float32),
            pltpu.VMEM((_K + 16,), jnp.float32),
            pltpu.VMEM((_K, _NH), jnp.float32),
        ] + 6 * [pltpu.SemaphoreType.DMA],
    )


def _alpha_pass(e2, den, srcH):
    def body(eH, denH, srcH_, aO,
             srcbA, ebufA, dbufA, abufA, srcbB, ebufB, dbufB, abufB,
             semGA, semGB, semSA, semSB):
        c = lax.axis_index("c")
        s = lax.axis_index("s")
        wid = s * _NC + c
        nch_mine = lax.select(wid < _NCH % _NW,
                              jnp.int32(_NCH // _NW + 1), jnp.int32(_NCH // _NW))
        nch2 = (nch_mine + 1) // 2

        def load(j, srcb, ebuf, sem):
            pltpu.sync_copy(srcH_.at[pl.ds(j * _K, _K)], srcb)
            pltpu.sync_copy(eH.at[pl.ds(j * _K, _K)], ebuf)
            pltpu.async_copy(denH.at[srcb], dbufA if srcb is srcbA else dbufB,
                             sem)

        def drain_gather(dbuf, sem):
            pltpu.make_async_copy(denH.at[pl.ds(0, _K)], dbuf, sem).wait()

        def compute(ebuf, dbuf, abuf):
            def grp(g, carry2):
                sl = pl.ds(g * 16, 16)
                abuf[sl] = ebuf[sl] / dbuf[sl]
                return carry2

            lax.fori_loop(0, _K // 16, grp, 0)

        def issue_store(j, abuf, sem):
            pltpu.async_copy(abuf, aO.at[pl.ds(j * _K, _K)], sem)

        def drain_store(abuf, sem):
            pltpu.make_async_copy(abuf, aO.at[pl.ds(0, _K)], sem).wait()

        load(wid, srcbA, ebufA, semGA)

        def piter(k2, carry):
            k0 = 2 * k2
            j0 = wid + k0 * _NW
            j1 = j0 + _NW
            has1 = k0 + 1 < nch_mine

            @pl.when(has1)
            def _():
                @pl.when(k2 > 0)
                def _():
                    drain_store(abufB, semSB)
                load(j1, srcbB, ebufB, semGB)

            drain_gather(dbufA, semGA)
            compute(ebufA, dbufA, abufA)
            issue_store(j0, abufA, semSA)

            @pl.when(has1)
            def _():
                drain_gather(dbufB, semGB)
                compute(ebufB, dbufB, abufB)
                issue_store(j1, abufB, semSB)

            @pl.when(k0 + 2 < nch_mine)
            def _():
                drain_store(abufA, semSA)
                load(j0 + 2 * _NW, srcbA, ebufA, semGA)

            return carry

        lax.fori_loop(0, nch2, piter, 0)
        drain_store(abufA, semSA)
        drain_store(abufB, semSB)

    return pl.kernel(
        body,
        out_type=jax.ShapeDtypeStruct((_E,), jnp.float32),
        mesh=_get_mesh(),
        scratch_types=2 * [
            pltpu.VMEM((_K,), jnp.int32),
            pltpu.VMEM((_K,), jnp.float32),
            pltpu.VMEM((_K,), jnp.float32),
            pltpu.VMEM((_K,), jnp.float32),
        ] + 4 * [pltpu.SemaphoreType.DMA],
    )(e2, den, srcH)


def kernel(x, edge_index, W1, a1_w, a1_b, W2, a2_w, a2_b, fc_w, fc_b):
    src = edge_index[0]
    dst = edge_index[1]
    x_pad = jnp.pad(x, ((0, _NPAD - _N), (0, 0)))

    h1, ss1, sd1 = _first_stage(x_pad, W1, a1_w, a1_b)
    acc1, den1 = _make_agg(False)(
        h1, ss1.reshape(_NPAD), sd1.reshape(_NPAD), src, dst)
    h2, ss2, sd2 = _mid_stage(acc1, den1, W2, a2_w, a2_b)
    acc2, den2, e2 = _make_agg(True)(
        h2, ss2.reshape(_NPAD), sd2.reshape(_NPAD), src, dst)
    out, den2s = _final_stage(acc2, den2, fc_w, fc_b)
    alpha = _alpha_pass(e2, den2s.reshape(_NPAD), src)
    return out[:_N], alpha


# speedup vs baseline: 20.0578x; 1.3629x over previous
import functools

import jax
import jax.numpy as jnp
from jax import lax
from jax.experimental import pallas as pl
from jax.experimental.pallas import tpu as pltpu
from jax.experimental.pallas import tpu_sc as plsc

_N = 10000
_E = 320000
_NH = 128
_NCLASS = 64
_K = 128
_NCH = _E // _K
_NC = 2
_NS = 16
_NW = _NC * _NS
_NPAD = 10240
_RPT = _NPAD // _NS
_R = 1024


def _dense_tail(h, aw_ref, ab_ref, hp_ref, ss_ref, sd_ref):
    aw_v = aw_ref[...]
    s_src = jnp.dot(h, aw_v[0:128], preferred_element_type=jnp.float32) + ab_ref[0]
    s_dst = jnp.dot(h, aw_v[128:256], preferred_element_type=jnp.float32)
    hp_ref[...] = h
    ss_ref[...] = s_src
    sd_ref[...] = s_dst


_NODE_OUT = [
    jax.ShapeDtypeStruct((_NPAD, _NH), jnp.float32),
    jax.ShapeDtypeStruct((_NPAD, 1), jnp.float32),
    jax.ShapeDtypeStruct((_NPAD, 1), jnp.float32),
]
_NODE_SPECS = [
    pl.BlockSpec((_R, _NH), lambda i: (i, 0)),
    pl.BlockSpec((_R, 1), lambda i: (i, 0)),
    pl.BlockSpec((_R, 1), lambda i: (i, 0)),
]


def _first_stage(x_pad, W, aw, ab):
    def body(x_ref, w_ref, aw_ref, ab_ref, hp_ref, ss_ref, sd_ref):
        h = jnp.dot(x_ref[...], w_ref[...], preferred_element_type=jnp.float32)
        _dense_tail(h, aw_ref, ab_ref, hp_ref, ss_ref, sd_ref)

    return pl.pallas_call(
        body,
        grid=(_NPAD // _R,),
        in_specs=[
            pl.BlockSpec((_R, _NH), lambda i: (i, 0)),
            pl.BlockSpec((_NH, _NH), lambda i: (0, 0)),
            pl.BlockSpec((2 * _NH, 1), lambda i: (0, 0)),
            pl.BlockSpec(memory_space=pltpu.SMEM),
        ],
        out_specs=_NODE_SPECS,
        out_shape=_NODE_OUT,
    )(x_pad, W, aw, ab)


def _mid_stage(acc, den, W, aw, ab):
    def body(acc_ref, den_ref, w_ref, aw_ref, ab_ref, hp_ref, ss_ref, sd_ref):
        sacc = acc_ref[0] + acc_ref[1]
        den_v = (den_ref[0] + den_ref[1]).reshape(_R, 1)
        den_v = jnp.where(den_v > 0.0, den_v, 1.0)
        h_in = jnp.maximum(sacc / den_v, 0.0)
        h = jnp.dot(h_in, w_ref[...], preferred_element_type=jnp.float32)
        _dense_tail(h, aw_ref, ab_ref, hp_ref, ss_ref, sd_ref)

    return pl.pallas_call(
        body,
        grid=(_NPAD // _R,),
        in_specs=[
            pl.BlockSpec((2, _R, _NH), lambda i: (0, i, 0)),
            pl.BlockSpec((2, _R), lambda i: (0, i)),
            pl.BlockSpec((_NH, _NH), lambda i: (0, 0)),
            pl.BlockSpec((2 * _NH, 1), lambda i: (0, 0)),
            pl.BlockSpec(memory_space=pltpu.SMEM),
        ],
        out_specs=_NODE_SPECS,
        out_shape=_NODE_OUT,
    )(acc, den, W, aw, ab)


def _final_stage(acc, den, fc_w, fc_b):
    def body(acc_ref, den_ref, w_ref, b_ref, out_ref, den_o_ref):
        sacc = acc_ref[0] + acc_ref[1]
        den_v = (den_ref[0] + den_ref[1]).reshape(_R, 1)
        den_s = jnp.where(den_v > 0.0, den_v, 1.0)
        h = jnp.maximum(sacc / den_s, 0.0)
        logits = jnp.dot(h, w_ref[...], preferred_element_type=jnp.float32) + b_ref[...]
        m = jnp.max(logits, axis=1, keepdims=True)
        lse = m + jnp.log(jnp.sum(jnp.exp(logits - m), axis=1, keepdims=True))
        out_ref[...] = logits - lse
        den_o_ref[...] = den_s

    return pl.pallas_call(
        body,
        grid=(_NPAD // _R,),
        in_specs=[
            pl.BlockSpec((2, _R, _NH), lambda i: (0, i, 0)),
            pl.BlockSpec((2, _R), lambda i: (0, i)),
            pl.BlockSpec((_NH, _NCLASS), lambda i: (0, 0)),
            pl.BlockSpec((1, _NCLASS), lambda i: (0, 0)),
        ],
        out_specs=[
            pl.BlockSpec((_R, _NCLASS), lambda i: (i, 0)),
            pl.BlockSpec((_R, 1), lambda i: (i, 0)),
        ],
        out_shape=[
            jax.ShapeDtypeStruct((_NPAD, _NCLASS), jnp.float32),
            jax.ShapeDtypeStruct((_NPAD, 1), jnp.float32),
        ],
    )(acc, den, fc_w, fc_b.reshape(1, _NCLASS))


@functools.cache
def _get_mesh():
    return plsc.VectorSubcoreMesh(core_axis_name="c", subcore_axis_name="s",
                                  num_cores=_NC, num_subcores=_NS)


@functools.cache
def _make_agg(want_e):
    out_type = [
        jax.ShapeDtypeStruct((_NC, _NPAD, _NH), jnp.float32),
        jax.ShapeDtypeStruct((_NC, _NPAD), jnp.float32),
    ]
    if want_e:
        out_type.append(jax.ShapeDtypeStruct((_E,), jnp.float32))

    def body(hT, ssT, sdT, srcH, dstH, *rest):
        if want_e:
            (accO, denO, eO, acc_sh, den_sh,
             srcbA, dstbA, ssrcbA, sdstbA, ebufA, rowbufA, srcsA,
             srcbB, dstbB, ssrcbB, sdstbB, ebufB, rowbufB, srcsB,
             semGA, semGB, semSA, semSB, semEA, semEB, semIA, semIB) = rest
        else:
            (accO, denO, acc_sh, den_sh,
             srcbA, dstbA, ssrcbA, sdstbA, ebufA, rowbufA, srcsA,
             srcbB, dstbB, ssrcbB, sdstbB, ebufB, rowbufB, srcsB,
             semGA, semGB, semSA, semSB, semEA, semEB, semIA, semIB) = rest
            eO = None
        c = lax.axis_index("c")
        s = lax.axis_index("s")
        wid = s * _NC + c

        zv = jnp.zeros((16,), jnp.float32)

        def zrow(i, carry):
            for g in range(_NH // 16):
                rowbufA[i, pl.ds(g * 16, 16)] = zv
            return carry

        lax.fori_loop(0, _K, zrow, 0)

        def zden(t, carry):
            ebufA[pl.ds(t * 16, 16)] = zv
            return carry

        lax.fori_loop(0, _K // 16, zden, 0)

        base = s * _RPT
        for t in range(_RPT // _K):
            pltpu.sync_copy(rowbufA, acc_sh.at[pl.ds(base + t * _K, _K)])
            pltpu.sync_copy(ebufA, den_sh.at[pl.ds(base + t * _K, _K)])
        plsc.subcore_barrier()

        nch_mine = lax.select(wid < _NCH % _NW,
                              jnp.int32(_NCH // _NW + 1), jnp.int32(_NCH // _NW))
        nch2 = (nch_mine + 1) // 2

        def issue_idx(j, srcb, dstb, semI):
            pltpu.async_copy(srcH.at[pl.ds(j * _K, _K)], srcb, semI)
            pltpu.async_copy(dstH.at[pl.ds(j * _K, _K)], dstb, semI)

        def drain_idx(srcb, dstb, semI):
            pltpu.make_async_copy(srcH.at[pl.ds(0, _K)], srcb, semI).wait()
            pltpu.make_async_copy(dstH.at[pl.ds(0, _K)], dstb, semI).wait()

        def copy_scatter_idx(srcb, srcs):
            for g in range(_K // 16):
                srcs[pl.ds(g * 16, 16)] = srcb[pl.ds(g * 16, 16)]

        def issue_gathers(srcb, dstb, ssrcb, sdstb, rowbuf, sem):
            pltpu.async_copy(ssT.at[srcb], ssrcb, sem)
            pltpu.async_copy(sdT.at[dstb], sdstb, sem)
            pltpu.async_copy(hT.at[dstb], rowbuf, sem)

        def drain_gathers(ssrcb, sdstb, rowbuf, sem):
            pltpu.make_async_copy(ssT.at[pl.ds(0, _K)], ssrcb, sem).wait()
            pltpu.make_async_copy(sdT.at[pl.ds(0, _K)], sdstb, sem).wait()
            pltpu.make_async_copy(hT.at[pl.ds(0, _K)], rowbuf, sem).wait()

        def compute(ssrcb, sdstb, ebuf, rowbuf):
            def egrp(g, carry2):
                sl = pl.ds(g * 16, 16)
                z = ssrcb[sl] + sdstb[sl]
                z = jnp.maximum(z, 0.05 * z)
                ebuf[sl] = jnp.exp(z)
                return carry2

            lax.fori_loop(0, _K // 16, egrp, 0)

            def rscale16(g, carry2):
                ev16 = ebuf[pl.ds(g * 16, 16)]
                for l in range(16):
                    i = g * 16 + l
                    ev = ev16[l]
                    for q in range(_NH // 16):
                        rowbuf[i, pl.ds(q * 16, 16)] = (
                            rowbuf[i, pl.ds(q * 16, 16)] * ev)
                return carry2

            lax.fori_loop(0, _K // 16, rscale16, 0)

        def issue_scatters(j, srcs, ebuf, rowbuf, semS, semE):
            pltpu.async_copy(rowbuf, acc_sh.at[srcs], semS, add=True)
            pltpu.async_copy(ebuf, den_sh.at[srcs], semS, add=True)
            if want_e:
                pltpu.async_copy(ebuf, eO.at[pl.ds(j * _K, _K)], semE)

        def drain_scatters(ebuf, rowbuf, semS, semE):
            pltpu.make_async_copy(hT.at[pl.ds(0, _K)], rowbuf, semS).wait()
            pltpu.make_async_copy(ssT.at[pl.ds(0, _K)], ebuf, semS).wait()
            if want_e:
                pltpu.make_async_copy(ebuf, eO.at[pl.ds(0, _K)], semE).wait()

        issue_idx(wid, srcbA, dstbA, semIA)
        drain_idx(srcbA, dstbA, semIA)
        issue_gathers(srcbA, dstbA, ssrcbA, sdstbA, rowbufA, semGA)

        @pl.when(jnp.int32(1) < nch_mine)
        def _():
            issue_idx(wid + _NW, srcbB, dstbB, semIB)

        def piter(k2, carry):
            k0 = 2 * k2
            j0 = wid + k0 * _NW
            j1 = j0 + _NW
            has1 = k0 + 1 < nch_mine

            @pl.when(has1)
            def _():
                @pl.when(k2 > 0)
                def _():
                    drain_scatters(ebufB, rowbufB, semSB, semEB)
                drain_idx(srcbB, dstbB, semIB)
                issue_gathers(srcbB, dstbB, ssrcbB, sdstbB, rowbufB, semGB)

            drain_gathers(ssrcbA, sdstbA, rowbufA, semGA)
            copy_scatter_idx(srcbA, srcsA)

            @pl.when(k0 + 2 < nch_mine)
            def _():
                issue_idx(j0 + 2 * _NW, srcbA, dstbA, semIA)

            compute(ssrcbA, sdstbA, ebufA, rowbufA)
            issue_scatters(j0, srcsA, ebufA, rowbufA, semSA, semEA)

            @pl.when(has1)
            def _():
                drain_gathers(ssrcbB, sdstbB, rowbufB, semGB)
                copy_scatter_idx(srcbB, srcsB)

                @pl.when(k0 + 3 < nch_mine)
                def _():
                    issue_idx(j1 + 2 * _NW, srcbB, dstbB, semIB)

                compute(ssrcbB, sdstbB, ebufB, rowbufB)
                issue_scatters(j1, srcsB, ebufB, rowbufB, semSB, semEB)

            @pl.when(k0 + 2 < nch_mine)
            def _():
                drain_scatters(ebufA, rowbufA, semSA, semEA)
                drain_idx(srcbA, dstbA, semIA)
                issue_gathers(srcbA, dstbA, ssrcbA, sdstbA, rowbufA, semGA)

            return carry

        lax.fori_loop(0, nch2, piter, 0)
        drain_scatters(ebufA, rowbufA, semSA, semEA)
        drain_scatters(ebufB, rowbufB, semSB, semEB)
        plsc.subcore_barrier()

        pltpu.sync_copy(acc_sh.at[pl.ds(base, _RPT)],
                        accO.at[c, pl.ds(base, _RPT)])
        pltpu.sync_copy(den_sh.at[pl.ds(base, _RPT)],
                        denO.at[c, pl.ds(base, _RPT)])

    return pl.kernel(
        body,
        out_type=out_type,
        mesh=_get_mesh(),
        scratch_types=[
            pltpu.VMEM_SHARED((_NPAD, _NH), jnp.float32),
            pltpu.VMEM_SHARED((_NPAD,), jnp.float32),
        ] + 2 * [
            pltpu.VMEM((_K,), jnp.int32),
            pltpu.VMEM((_K,), jnp.int32),
            pltpu.VMEM((_K,), jnp.float32),
            pltpu.VMEM((_K,), jnp.float32),
            pltpu.VMEM((_K,), jnp.float32),
            pltpu.VMEM((_K, _NH), jnp.float32),
            pltpu.VMEM((_K,), jnp.int32),
        ] + 8 * [pltpu.SemaphoreType.DMA],
    )


def _alpha_pass(e2, den, srcH):
    def body(eH, denH, srcH_, aO,
             srcbA, ebufA, dbufA, abufA, srcbB, ebufB, dbufB, abufB,
             semGA, semGB, semSA, semSB, semIA, semIB):
        c = lax.axis_index("c")
        s = lax.axis_index("s")
        wid = s * _NC + c
        nch_mine = lax.select(wid < _NCH % _NW,
                              jnp.int32(_NCH // _NW + 1), jnp.int32(_NCH // _NW))
        nch2 = (nch_mine + 1) // 2

        def issue_idx(j, srcb, ebuf, semI):
            pltpu.async_copy(srcH_.at[pl.ds(j * _K, _K)], srcb, semI)
            pltpu.async_copy(eH.at[pl.ds(j * _K, _K)], ebuf, semI)

        def drain_idx(srcb, ebuf, semI):
            pltpu.make_async_copy(srcH_.at[pl.ds(0, _K)], srcb, semI).wait()
            pltpu.make_async_copy(eH.at[pl.ds(0, _K)], ebuf, semI).wait()

        def issue_gather(srcb, dbuf, sem):
            pltpu.async_copy(denH.at[srcb], dbuf, sem)

        def drain_gather(dbuf, sem):
            pltpu.make_async_copy(denH.at[pl.ds(0, _K)], dbuf, sem).wait()

        def compute(ebuf, dbuf, abuf):
            def grp(g, carry2):
                sl = pl.ds(g * 16, 16)
                abuf[sl] = ebuf[sl] / dbuf[sl]
                return carry2

            lax.fori_loop(0, _K // 16, grp, 0)

        def issue_store(j, abuf, sem):
            pltpu.async_copy(abuf, aO.at[pl.ds(j * _K, _K)], sem)

        def drain_store(abuf, sem):
            pltpu.make_async_copy(abuf, aO.at[pl.ds(0, _K)], sem).wait()

        issue_idx(wid, srcbA, ebufA, semIA)
        drain_idx(srcbA, ebufA, semIA)
        issue_gather(srcbA, dbufA, semGA)

        @pl.when(jnp.int32(1) < nch_mine)
        def _():
            issue_idx(wid + _NW, srcbB, ebufB, semIB)

        def piter(k2, carry):
            k0 = 2 * k2
            j0 = wid + k0 * _NW
            j1 = j0 + _NW
            has1 = k0 + 1 < nch_mine

            @pl.when(has1)
            def _():
                @pl.when(k2 > 0)
                def _():
                    drain_store(abufB, semSB)
                drain_idx(srcbB, ebufB, semIB)
                issue_gather(srcbB, dbufB, semGB)

            drain_gather(dbufA, semGA)
            compute(ebufA, dbufA, abufA)
            issue_store(j0, abufA, semSA)

            @pl.when(k0 + 2 < nch_mine)
            def _():
                issue_idx(j0 + 2 * _NW, srcbA, ebufA, semIA)

            @pl.when(has1)
            def _():
                drain_gather(dbufB, semGB)
                compute(ebufB, dbufB, abufB)
                issue_store(j1, abufB, semSB)

                @pl.when(k0 + 3 < nch_mine)
                def _():
                    issue_idx(j1 + 2 * _NW, srcbB, ebufB, semIB)

            @pl.when(k0 + 2 < nch_mine)
            def _():
                drain_store(abufA, semSA)
                drain_idx(srcbA, ebufA, semIA)
                issue_gather(srcbA, dbufA, semGA)

            return carry

        lax.fori_loop(0, nch2, piter, 0)
        drain_store(abufA, semSA)
        drain_store(abufB, semSB)

    return pl.kernel(
        body,
        out_type=jax.ShapeDtypeStruct((_E,), jnp.float32),
        mesh=_get_mesh(),
        scratch_types=2 * [
            pltpu.VMEM((_K,), jnp.int32),
            pltpu.VMEM((_K,), jnp.float32),
            pltpu.VMEM((_K,), jnp.float32),
            pltpu.VMEM((_K,), jnp.float32),
        ] + 6 * [pltpu.SemaphoreType.DMA],
    )(e2, den, srcH)


def kernel(x, edge_index, W1, a1_w, a1_b, W2, a2_w, a2_b, fc_w, fc_b):
    src = edge_index[0]
    dst = edge_index[1]
    x_pad = jnp.pad(x, ((0, _NPAD - _N), (0, 0)))

    h1, ss1, sd1 = _first_stage(x_pad, W1, a1_w, a1_b)
    acc1, den1 = _make_agg(False)(
        h1, ss1.reshape(_NPAD), sd1.reshape(_NPAD), src, dst)
    h2, ss2, sd2 = _mid_stage(acc1, den1, W2, a2_w, a2_b)
    acc2, den2, e2 = _make_agg(True)(
        h2, ss2.reshape(_NPAD), sd2.reshape(_NPAD), src, dst)
    out, den2s = _final_stage(acc2, den2, fc_w, fc_b)
    alpha = _alpha_pass(e2, den2s.reshape(_NPAD), src)
    return out[:_N], alpha


# speedup vs baseline: 20.1622x; 1.0052x over previous
import functools

import jax
import jax.numpy as jnp
from jax import lax
from jax.experimental import pallas as pl
from jax.experimental.pallas import tpu as pltpu
from jax.experimental.pallas import tpu_sc as plsc

_N = 10000
_E = 320000
_NH = 128
_NCLASS = 64
_K = 128
_NCH = _E // _K
_NC = 2
_NS = 16
_NW = _NC * _NS
_NPAD = 10240
_RPT = _NPAD // _NS
_R = 1024


def _dense_tail(h, aw_ref, ab_ref, hp_ref, ss_ref, sd_ref):
    aw_v = aw_ref[...]
    s_src = jnp.dot(h, aw_v[0:128], preferred_element_type=jnp.float32) + ab_ref[0]
    s_dst = jnp.dot(h, aw_v[128:256], preferred_element_type=jnp.float32)
    hp_ref[...] = h
    ss_ref[...] = s_src
    sd_ref[...] = s_dst


_NODE_OUT = [
    jax.ShapeDtypeStruct((_NPAD, _NH), jnp.float32),
    jax.ShapeDtypeStruct((_NPAD, 1), jnp.float32),
    jax.ShapeDtypeStruct((_NPAD, 1), jnp.float32),
]


def _node_specs(R):
    return [
        pl.BlockSpec((R, _NH), lambda i: (i, 0)),
        pl.BlockSpec((R, 1), lambda i: (i, 0)),
        pl.BlockSpec((R, 1), lambda i: (i, 0)),
    ]


def _first_stage(x, W, aw, ab, R=1000):
    def body(x_ref, w_ref, aw_ref, ab_ref, hp_ref, ss_ref, sd_ref):
        h = jnp.dot(x_ref[...], w_ref[...], preferred_element_type=jnp.float32)
        _dense_tail(h, aw_ref, ab_ref, hp_ref, ss_ref, sd_ref)

    return pl.pallas_call(
        body,
        grid=(_N // R,),
        in_specs=[
            pl.BlockSpec((R, _NH), lambda i: (i, 0)),
            pl.BlockSpec((_NH, _NH), lambda i: (0, 0)),
            pl.BlockSpec((2 * _NH, 1), lambda i: (0, 0)),
            pl.BlockSpec(memory_space=pltpu.SMEM),
        ],
        out_specs=_node_specs(R),
        out_shape=_NODE_OUT,
    )(x, W, aw, ab)


def _mid_stage(acc, den, W, aw, ab):
    def body(acc_ref, den_ref, w_ref, aw_ref, ab_ref, hp_ref, ss_ref, sd_ref):
        sacc = acc_ref[0] + acc_ref[1]
        den_v = (den_ref[0] + den_ref[1]).reshape(_R, 1)
        den_v = jnp.where(den_v > 0.0, den_v, 1.0)
        h_in = jnp.maximum(sacc / den_v, 0.0)
        h = jnp.dot(h_in, w_ref[...], preferred_element_type=jnp.float32)
        _dense_tail(h, aw_ref, ab_ref, hp_ref, ss_ref, sd_ref)

    return pl.pallas_call(
        body,
        grid=(_NPAD // _R,),
        in_specs=[
            pl.BlockSpec((2, _R, _NH), lambda i: (0, i, 0)),
            pl.BlockSpec((2, _R), lambda i: (0, i)),
            pl.BlockSpec((_NH, _NH), lambda i: (0, 0)),
            pl.BlockSpec((2 * _NH, 1), lambda i: (0, 0)),
            pl.BlockSpec(memory_space=pltpu.SMEM),
        ],
        out_specs=_node_specs(_R),
        out_shape=_NODE_OUT,
    )(acc, den, W, aw, ab)


def _final_stage(acc, den, fc_w, fc_b):
    def body(acc_ref, den_ref, w_ref, b_ref, out_ref):
        sacc = acc_ref[0] + acc_ref[1]
        den_v = (den_ref[0] + den_ref[1]).reshape(_R, 1)
        den_s = jnp.where(den_v > 0.0, den_v, 1.0)
        h = jnp.maximum(sacc / den_s, 0.0)
        logits = jnp.dot(h, w_ref[...], preferred_element_type=jnp.float32) + b_ref[...]
        m = jnp.max(logits, axis=1, keepdims=True)
        lse = m + jnp.log(jnp.sum(jnp.exp(logits - m), axis=1, keepdims=True))
        out_ref[...] = logits - lse

    return pl.pallas_call(
        body,
        grid=(_NPAD // _R,),
        in_specs=[
            pl.BlockSpec((2, _R, _NH), lambda i: (0, i, 0)),
            pl.BlockSpec((2, _R), lambda i: (0, i)),
            pl.BlockSpec((_NH, _NCLASS), lambda i: (0, 0)),
            pl.BlockSpec((1, _NCLASS), lambda i: (0, 0)),
        ],
        out_specs=pl.BlockSpec((_R, _NCLASS), lambda i: (i, 0)),
        out_shape=jax.ShapeDtypeStruct((_NPAD, _NCLASS), jnp.float32),
    )(acc, den, fc_w, fc_b.reshape(1, _NCLASS))


@functools.cache
def _get_mesh():
    return plsc.VectorSubcoreMesh(core_axis_name="c", subcore_axis_name="s",
                                  num_cores=_NC, num_subcores=_NS)


@functools.cache
def _make_agg(want_e):
    out_type = [
        jax.ShapeDtypeStruct((_NC, _NPAD, _NH), jnp.float32),
        jax.ShapeDtypeStruct((_NC, _NPAD), jnp.float32),
    ]
    if want_e:
        out_type.append(jax.ShapeDtypeStruct((_E,), jnp.float32))

    def body(hT, ssT, sdT, srcH, dstH, *rest):
        if want_e:
            (accO, denO, eO, acc_sh, den_sh,
             srcbA, dstbA, ssrcbA, sdstbA, ebufA, rowbufA, srcsA,
             srcbB, dstbB, ssrcbB, sdstbB, ebufB, rowbufB, srcsB,
             semGA, semGB, semSA, semSB, semEA, semEB, semIA, semIB) = rest
        else:
            (accO, denO, acc_sh, den_sh,
             srcbA, dstbA, ssrcbA, sdstbA, ebufA, rowbufA, srcsA,
             srcbB, dstbB, ssrcbB, sdstbB, ebufB, rowbufB, srcsB,
             semGA, semGB, semSA, semSB, semEA, semEB, semIA, semIB) = rest
            eO = None
        c = lax.axis_index("c")
        s = lax.axis_index("s")
        wid = s * _NC + c

        zv = jnp.zeros((16,), jnp.float32)

        def zrow(i, carry):
            for g in range(_NH // 16):
                rowbufA[i, pl.ds(g * 16, 16)] = zv
            return carry

        lax.fori_loop(0, _K, zrow, 0)

        def zden(t, carry):
            ebufA[pl.ds(t * 16, 16)] = zv
            return carry

        lax.fori_loop(0, _K // 16, zden, 0)

        base = s * _RPT
        for t in range(_RPT // _K):
            pltpu.sync_copy(rowbufA, acc_sh.at[pl.ds(base + t * _K, _K)])
            pltpu.sync_copy(ebufA, den_sh.at[pl.ds(base + t * _K, _K)])
        plsc.subcore_barrier()

        nch_mine = lax.select(wid < _NCH % _NW,
                              jnp.int32(_NCH // _NW + 1), jnp.int32(_NCH // _NW))
        nch2 = (nch_mine + 1) // 2

        def issue_idx(j, srcb, dstb, semI):
            pltpu.async_copy(srcH.at[pl.ds(j * _K, _K)], srcb, semI)
            pltpu.async_copy(dstH.at[pl.ds(j * _K, _K)], dstb, semI)

        def drain_idx(srcb, dstb, semI):
            pltpu.make_async_copy(srcH.at[pl.ds(0, _K)], srcb, semI).wait()
            pltpu.make_async_copy(dstH.at[pl.ds(0, _K)], dstb, semI).wait()

        def copy_scatter_idx(srcb, srcs):
            for g in range(_K // 16):
                srcs[pl.ds(g * 16, 16)] = srcb[pl.ds(g * 16, 16)]

        def issue_gathers(srcb, dstb, ssrcb, sdstb, rowbuf, sem):
            pltpu.async_copy(ssT.at[srcb], ssrcb, sem)
            pltpu.async_copy(sdT.at[dstb], sdstb, sem)
            pltpu.async_copy(hT.at[dstb], rowbuf, sem)

        def drain_gathers(ssrcb, sdstb, rowbuf, sem):
            pltpu.make_async_copy(ssT.at[pl.ds(0, _K)], ssrcb, sem).wait()
            pltpu.make_async_copy(sdT.at[pl.ds(0, _K)], sdstb, sem).wait()
            pltpu.make_async_copy(hT.at[pl.ds(0, _K)], rowbuf, sem).wait()

        def compute(ssrcb, sdstb, ebuf, rowbuf):
            def egrp(g, carry2):
                sl = pl.ds(g * 16, 16)
                z = ssrcb[sl] + sdstb[sl]
                z = jnp.maximum(z, 0.05 * z)
                ebuf[sl] = jnp.exp(z)
                return carry2

            lax.fori_loop(0, _K // 16, egrp, 0)

            def rscale16(g, carry2):
                ev16 = ebuf[pl.ds(g * 16, 16)]
                for l in range(16):
                    i = g * 16 + l
                    ev = ev16[l]
                    for q in range(_NH // 16):
                        rowbuf[i, pl.ds(q * 16, 16)] = (
                            rowbuf[i, pl.ds(q * 16, 16)] * ev)
                return carry2

            lax.fori_loop(0, _K // 16, rscale16, 0)

        def issue_scatters(j, srcs, ebuf, rowbuf, semS, semE):
            pltpu.async_copy(rowbuf, acc_sh.at[srcs], semS, add=True)
            pltpu.async_copy(ebuf, den_sh.at[srcs], semS, add=True)
            if want_e:
                pltpu.async_copy(ebuf, eO.at[pl.ds(j * _K, _K)], semE)

        def drain_scatters(ebuf, rowbuf, semS, semE):
            pltpu.make_async_copy(hT.at[pl.ds(0, _K)], rowbuf, semS).wait()
            pltpu.make_async_copy(ssT.at[pl.ds(0, _K)], ebuf, semS).wait()
            if want_e:
                pltpu.make_async_copy(ebuf, eO.at[pl.ds(0, _K)], semE).wait()

        issue_idx(wid, srcbA, dstbA, semIA)
        drain_idx(srcbA, dstbA, semIA)
        issue_gathers(srcbA, dstbA, ssrcbA, sdstbA, rowbufA, semGA)

        @pl.when(jnp.int32(1) < nch_mine)
        def _():
            issue_idx(wid + _NW, srcbB, dstbB, semIB)

        def piter(k2, carry):
            k0 = 2 * k2
            j0 = wid + k0 * _NW
            j1 = j0 + _NW
            has1 = k0 + 1 < nch_mine

            @pl.when(has1)
            def _():
                @pl.when(k2 > 0)
                def _():
                    drain_scatters(ebufB, rowbufB, semSB, semEB)
                drain_idx(srcbB, dstbB, semIB)
                issue_gathers(srcbB, dstbB, ssrcbB, sdstbB, rowbufB, semGB)

            drain_gathers(ssrcbA, sdstbA, rowbufA, semGA)
            copy_scatter_idx(srcbA, srcsA)

            @pl.when(k0 + 2 < nch_mine)
            def _():
                issue_idx(j0 + 2 * _NW, srcbA, dstbA, semIA)

            compute(ssrcbA, sdstbA, ebufA, rowbufA)
            issue_scatters(j0, srcsA, ebufA, rowbufA, semSA, semEA)

            @pl.when(has1)
            def _():
                drain_gathers(ssrcbB, sdstbB, rowbufB, semGB)
                copy_scatter_idx(srcbB, srcsB)

                @pl.when(k0 + 3 < nch_mine)
                def _():
                    issue_idx(j1 + 2 * _NW, srcbB, dstbB, semIB)

                compute(ssrcbB, sdstbB, ebufB, rowbufB)
                issue_scatters(j1, srcsB, ebufB, rowbufB, semSB, semEB)

            @pl.when(k0 + 2 < nch_mine)
            def _():
                drain_scatters(ebufA, rowbufA, semSA, semEA)
                drain_idx(srcbA, dstbA, semIA)
                issue_gathers(srcbA, dstbA, ssrcbA, sdstbA, rowbufA, semGA)

            return carry

        lax.fori_loop(0, nch2, piter, 0)
        drain_scatters(ebufA, rowbufA, semSA, semEA)
        drain_scatters(ebufB, rowbufB, semSB, semEB)
        plsc.subcore_barrier()

        pltpu.sync_copy(acc_sh.at[pl.ds(base, _RPT)],
                        accO.at[c, pl.ds(base, _RPT)])
        pltpu.sync_copy(den_sh.at[pl.ds(base, _RPT)],
                        denO.at[c, pl.ds(base, _RPT)])

    return pl.kernel(
        body,
        out_type=out_type,
        mesh=_get_mesh(),
        scratch_types=[
            pltpu.VMEM_SHARED((_NPAD, _NH), jnp.float32),
            pltpu.VMEM_SHARED((_NPAD,), jnp.float32),
        ] + 2 * [
            pltpu.VMEM((_K,), jnp.int32),
            pltpu.VMEM((_K,), jnp.int32),
            pltpu.VMEM((_K,), jnp.float32),
            pltpu.VMEM((_K,), jnp.float32),
            pltpu.VMEM((_K,), jnp.float32),
            pltpu.VMEM((_K, _NH), jnp.float32),
            pltpu.VMEM((_K,), jnp.int32),
        ] + 8 * [pltpu.SemaphoreType.DMA],
    )


def _alpha_pass(e2, den0, den1, srcH):
    def body(eH, den0H, den1H, srcH_, aO,
             srcbA, ebufA, dbufA, d1bufA, abufA,
             srcbB, ebufB, dbufB, d1bufB, abufB,
             semGA, semGB, semSA, semSB, semIA, semIB):
        c = lax.axis_index("c")
        s = lax.axis_index("s")
        wid = s * _NC + c
        nch_mine = lax.select(wid < _NCH % _NW,
                              jnp.int32(_NCH // _NW + 1), jnp.int32(_NCH // _NW))
        nch2 = (nch_mine + 1) // 2

        def issue_idx(j, srcb, ebuf, semI):
            pltpu.async_copy(srcH_.at[pl.ds(j * _K, _K)], srcb, semI)
            pltpu.async_copy(eH.at[pl.ds(j * _K, _K)], ebuf, semI)

        def drain_idx(srcb, ebuf, semI):
            pltpu.make_async_copy(srcH_.at[pl.ds(0, _K)], srcb, semI).wait()
            pltpu.make_async_copy(eH.at[pl.ds(0, _K)], ebuf, semI).wait()

        def issue_gather(srcb, dbuf, d1buf, sem):
            pltpu.async_copy(den0H.at[srcb], dbuf, sem)
            pltpu.async_copy(den1H.at[srcb], d1buf, sem)

        def drain_gather(dbuf, d1buf, sem):
            pltpu.make_async_copy(den0H.at[pl.ds(0, _K)], dbuf, sem).wait()
            pltpu.make_async_copy(den1H.at[pl.ds(0, _K)], d1buf, sem).wait()

        def compute(ebuf, dbuf, d1buf, abuf):
            def grp(g, carry2):
                sl = pl.ds(g * 16, 16)
                abuf[sl] = ebuf[sl] / (dbuf[sl] + d1buf[sl])
                return carry2

            lax.fori_loop(0, _K // 16, grp, 0)

        def issue_store(j, abuf, sem):
            pltpu.async_copy(abuf, aO.at[pl.ds(j * _K, _K)], sem)

        def drain_store(abuf, sem):
            pltpu.make_async_copy(abuf, aO.at[pl.ds(0, _K)], sem).wait()

        issue_idx(wid, srcbA, ebufA, semIA)
        drain_idx(srcbA, ebufA, semIA)
        issue_gather(srcbA, dbufA, d1bufA, semGA)

        @pl.when(jnp.int32(1) < nch_mine)
        def _():
            issue_idx(wid + _NW, srcbB, ebufB, semIB)

        def piter(k2, carry):
            k0 = 2 * k2
            j0 = wid + k0 * _NW
            j1 = j0 + _NW
            has1 = k0 + 1 < nch_mine

            @pl.when(has1)
            def _():
                @pl.when(k2 > 0)
                def _():
                    drain_store(abufB, semSB)
                drain_idx(srcbB, ebufB, semIB)
                issue_gather(srcbB, dbufB, d1bufB, semGB)

            drain_gather(dbufA, d1bufA, semGA)
            compute(ebufA, dbufA, d1bufA, abufA)
            issue_store(j0, abufA, semSA)

            @pl.when(k0 + 2 < nch_mine)
            def _():
                issue_idx(j0 + 2 * _NW, srcbA, ebufA, semIA)

            @pl.when(has1)
            def _():
                drain_gather(dbufB, d1bufB, semGB)
                compute(ebufB, dbufB, d1bufB, abufB)
                issue_store(j1, abufB, semSB)

                @pl.when(k0 + 3 < nch_mine)
                def _():
                    issue_idx(j1 + 2 * _NW, srcbB, ebufB, semIB)

            @pl.when(k0 + 2 < nch_mine)
            def _():
                drain_store(abufA, semSA)
                drain_idx(srcbA, ebufA, semIA)
                issue_gather(srcbA, dbufA, d1bufA, semGA)

            return carry

        lax.fori_loop(0, nch2, piter, 0)
        drain_store(abufA, semSA)
        drain_store(abufB, semSB)

    return pl.kernel(
        body,
        out_type=jax.ShapeDtypeStruct((_E,), jnp.float32),
        mesh=_get_mesh(),
        scratch_types=2 * [
            pltpu.VMEM((_K,), jnp.int32),
            pltpu.VMEM((_K,), jnp.float32),
            pltpu.VMEM((_K,), jnp.float32),
            pltpu.VMEM((_K,), jnp.float32),
            pltpu.VMEM((_K,), jnp.float32),
        ] + 6 * [pltpu.SemaphoreType.DMA],
    )(e2, den0, den1, srcH)


def kernel(x, edge_index, W1, a1_w, a1_b, W2, a2_w, a2_b, fc_w, fc_b):
    src = edge_index[0]
    dst = edge_index[1]

    h1, ss1, sd1 = _first_stage(x, W1, a1_w, a1_b)
    acc1, den1 = _make_agg(False)(
        h1, ss1.reshape(_NPAD), sd1.reshape(_NPAD), src, dst)
    h2, ss2, sd2 = _mid_stage(acc1, den1, W2, a2_w, a2_b)
    acc2, den2, e2 = _make_agg(True)(
        h2, ss2.reshape(_NPAD), sd2.reshape(_NPAD), src, dst)
    out = _final_stage(acc2, den2, fc_w, fc_b)
    alpha = _alpha_pass(e2, den2[0], den2[1], src)
    return out[:_N], alpha


# speedup vs baseline: 20.4505x; 1.0143x over previous
import functools

import jax
import jax.numpy as jnp
from jax import lax
from jax.experimental import pallas as pl
from jax.experimental.pallas import tpu as pltpu
from jax.experimental.pallas import tpu_sc as plsc

_N = 10000
_E = 320000
_NH = 128
_NCLASS = 64
_K = 128
_KC = 160
_HALVES = ((0, 128), (128, 32))
_NCH = _E // _K
_NCHC = _E // _KC
_NC = 2
_NS = 16
_NW = _NC * _NS
_NPAD = 10240
_RPT = _NPAD // _NS
_R = 1024


def _dense_tail(h, aw_ref, ab_ref, hp_ref, ss_ref, sd_ref):
    aw_v = aw_ref[...]
    s_src = jnp.dot(h, aw_v[0:128], preferred_element_type=jnp.float32) + ab_ref[0]
    s_dst = jnp.dot(h, aw_v[128:256], preferred_element_type=jnp.float32)
    hp_ref[...] = h
    ss_ref[...] = s_src
    sd_ref[...] = s_dst


_NODE_OUT = [
    jax.ShapeDtypeStruct((_NPAD, _NH), jnp.float32),
    jax.ShapeDtypeStruct((_NPAD, 1), jnp.float32),
    jax.ShapeDtypeStruct((_NPAD, 1), jnp.float32),
]


def _node_specs(R):
    return [
        pl.BlockSpec((R, _NH), lambda i: (i, 0)),
        pl.BlockSpec((R, 1), lambda i: (i, 0)),
        pl.BlockSpec((R, 1), lambda i: (i, 0)),
    ]


def _first_stage(x, W, aw, ab, R=1000):
    def body(x_ref, w_ref, aw_ref, ab_ref, hp_ref, ss_ref, sd_ref):
        h = jnp.dot(x_ref[...], w_ref[...], preferred_element_type=jnp.float32)
        _dense_tail(h, aw_ref, ab_ref, hp_ref, ss_ref, sd_ref)

    return pl.pallas_call(
        body,
        grid=(_N // R,),
        in_specs=[
            pl.BlockSpec((R, _NH), lambda i: (i, 0)),
            pl.BlockSpec((_NH, _NH), lambda i: (0, 0)),
            pl.BlockSpec((2 * _NH, 1), lambda i: (0, 0)),
            pl.BlockSpec(memory_space=pltpu.SMEM),
        ],
        out_specs=_node_specs(R),
        out_shape=_NODE_OUT,
    )(x, W, aw, ab)


def _mid_stage(acc, den, W, aw, ab):
    def body(acc_ref, den_ref, w_ref, aw_ref, ab_ref, hp_ref, ss_ref, sd_ref):
        sacc = acc_ref[0] + acc_ref[1]
        den_v = (den_ref[0] + den_ref[1]).reshape(_R, 1)
        den_v = jnp.where(den_v > 0.0, den_v, 1.0)
        h_in = jnp.maximum(sacc / den_v, 0.0)
        h = jnp.dot(h_in, w_ref[...], preferred_element_type=jnp.float32)
        _dense_tail(h, aw_ref, ab_ref, hp_ref, ss_ref, sd_ref)

    return pl.pallas_call(
        body,
        grid=(_NPAD // _R,),
        in_specs=[
            pl.BlockSpec((2, _R, _NH), lambda i: (0, i, 0)),
            pl.BlockSpec((2, _R), lambda i: (0, i)),
            pl.BlockSpec((_NH, _NH), lambda i: (0, 0)),
            pl.BlockSpec((2 * _NH, 1), lambda i: (0, 0)),
            pl.BlockSpec(memory_space=pltpu.SMEM),
        ],
        out_specs=_node_specs(_R),
        out_shape=_NODE_OUT,
    )(acc, den, W, aw, ab)


def _final_stage(acc, den, fc_w, fc_b):
    def body(acc_ref, den_ref, w_ref, b_ref, out_ref):
        sacc = acc_ref[0] + acc_ref[1]
        den_v = (den_ref[0] + den_ref[1]).reshape(_R, 1)
        den_s = jnp.where(den_v > 0.0, den_v, 1.0)
        h = jnp.maximum(sacc / den_s, 0.0)
        logits = jnp.dot(h, w_ref[...], preferred_element_type=jnp.float32) + b_ref[...]
        m = jnp.max(logits, axis=1, keepdims=True)
        lse = m + jnp.log(jnp.sum(jnp.exp(logits - m), axis=1, keepdims=True))
        out_ref[...] = logits - lse

    return pl.pallas_call(
        body,
        grid=(_NPAD // _R,),
        in_specs=[
            pl.BlockSpec((2, _R, _NH), lambda i: (0, i, 0)),
            pl.BlockSpec((2, _R), lambda i: (0, i)),
            pl.BlockSpec((_NH, _NCLASS), lambda i: (0, 0)),
            pl.BlockSpec((1, _NCLASS), lambda i: (0, 0)),
        ],
        out_specs=pl.BlockSpec((_R, _NCLASS), lambda i: (i, 0)),
        out_shape=jax.ShapeDtypeStruct((_NPAD, _NCLASS), jnp.float32),
    )(acc, den, fc_w, fc_b.reshape(1, _NCLASS))


@functools.cache
def _get_mesh():
    return plsc.VectorSubcoreMesh(core_axis_name="c", subcore_axis_name="s",
                                  num_cores=_NC, num_subcores=_NS)


@functools.cache
def _make_agg(want_e):
    out_type = [
        jax.ShapeDtypeStruct((_NC, _NPAD, _NH), jnp.float32),
        jax.ShapeDtypeStruct((_NC, _NPAD), jnp.float32),
    ]
    if want_e:
        out_type.append(jax.ShapeDtypeStruct((_E,), jnp.float32))

    def body(hT, ssT, sdT, srcH, dstH, *rest):
        if want_e:
            (accO, denO, eO, acc_sh, den_sh,
             srcbA, dstbA, ssrcbA, sdstbA, ebufA, rowbufA, srcs0A, srcs1A,
             srcbB, dstbB, ssrcbB, sdstbB, ebufB, rowbufB, srcs0B, srcs1B,
             semGA, semGB, semSA, semSB, semEA, semEB, semIA, semIB) = rest
        else:
            (accO, denO, acc_sh, den_sh,
             srcbA, dstbA, ssrcbA, sdstbA, ebufA, rowbufA, srcs0A, srcs1A,
             srcbB, dstbB, ssrcbB, sdstbB, ebufB, rowbufB, srcs0B, srcs1B,
             semGA, semGB, semSA, semSB, semEA, semEB, semIA, semIB) = rest
            eO = None
        c = lax.axis_index("c")
        s = lax.axis_index("s")
        wid = s * _NC + c

        zv = jnp.zeros((16,), jnp.float32)

        def zrow(i, carry):
            for g in range(_NH // 16):
                rowbufA[i, pl.ds(g * 16, 16)] = zv
            return carry

        lax.fori_loop(0, _KC, zrow, 0)

        def zden(t, carry):
            ebufA[pl.ds(t * 16, 16)] = zv
            return carry

        lax.fori_loop(0, _KC // 16, zden, 0)

        base = s * _RPT
        for t in range(_RPT // _KC):
            pltpu.sync_copy(rowbufA, acc_sh.at[pl.ds(base + t * _KC, _KC)])
            pltpu.sync_copy(ebufA, den_sh.at[pl.ds(base + t * _KC, _KC)])
        rem = _RPT % _KC
        if rem:
            off = base + (_RPT // _KC) * _KC
            pltpu.sync_copy(rowbufA.at[pl.ds(0, rem)],
                            acc_sh.at[pl.ds(off, rem)])
            pltpu.sync_copy(ebufA.at[pl.ds(0, rem)],
                            den_sh.at[pl.ds(off, rem)])
        plsc.subcore_barrier()

        nch_mine = lax.select(wid < _NCHC % _NW,
                              jnp.int32(_NCHC // _NW + 1),
                              jnp.int32(_NCHC // _NW))
        nch2 = (nch_mine + 1) // 2

        def issue_idx(j, srcb, dstb, semI):
            pltpu.async_copy(srcH.at[pl.ds(j * _KC, _KC)], srcb, semI)
            pltpu.async_copy(dstH.at[pl.ds(j * _KC, _KC)], dstb, semI)

        def drain_idx(srcb, dstb, semI):
            pltpu.make_async_copy(srcH.at[pl.ds(0, _KC)], srcb, semI).wait()
            pltpu.make_async_copy(dstH.at[pl.ds(0, _KC)], dstb, semI).wait()

        def copy_scatter_idx(srcb, srcs0, srcs1):
            for g in range(_HALVES[0][1] // 16):
                srcs0[pl.ds(g * 16, 16)] = srcb[pl.ds(g * 16, 16)]
            for g in range(_HALVES[1][1] // 16):
                srcs1[pl.ds(g * 16, 16)] = srcb[pl.ds(_HALVES[1][0] + g * 16,
                                                      16)]

        def issue_gathers(srcb, dstb, ssrcb, sdstb, rowbuf, sem):
            for off, sz in _HALVES:
                sl = pl.ds(off, sz)
                pltpu.async_copy(ssT.at[srcb.at[sl]], ssrcb.at[sl], sem)
                pltpu.async_copy(sdT.at[dstb.at[sl]], sdstb.at[sl], sem)
                pltpu.async_copy(hT.at[dstb.at[sl]], rowbuf.at[sl], sem)

        def drain_gathers(ssrcb, sdstb, rowbuf, sem):
            for off, sz in _HALVES:
                sl = pl.ds(off, sz)
                pltpu.make_async_copy(ssT.at[pl.ds(0, sz)], ssrcb.at[sl],
                                      sem).wait()
                pltpu.make_async_copy(sdT.at[pl.ds(0, sz)], sdstb.at[sl],
                                      sem).wait()
                pltpu.make_async_copy(hT.at[pl.ds(0, sz)], rowbuf.at[sl],
                                      sem).wait()

        def compute(ssrcb, sdstb, ebuf, rowbuf):
            def egrp(g, carry2):
                sl = pl.ds(g * 16, 16)
                z = ssrcb[sl] + sdstb[sl]
                z = jnp.maximum(z, 0.05 * z)
                ebuf[sl] = jnp.exp(z)
                return carry2

            lax.fori_loop(0, _KC // 16, egrp, 0)

            def rscale16(g, carry2):
                ev16 = ebuf[pl.ds(g * 16, 16)]
                for l in range(16):
                    i = g * 16 + l
                    ev = ev16[l]
                    for q in range(_NH // 16):
                        rowbuf[i, pl.ds(q * 16, 16)] = (
                            rowbuf[i, pl.ds(q * 16, 16)] * ev)
                return carry2

            lax.fori_loop(0, _KC // 16, rscale16, 0)

        def issue_scatters(j, srcs0, srcs1, ebuf, rowbuf, semS, semE):
            for (off, sz), srcs in zip(_HALVES, (srcs0, srcs1)):
                sl = pl.ds(off, sz)
                pltpu.async_copy(rowbuf.at[sl], acc_sh.at[srcs], semS,
                                 add=True)
                pltpu.async_copy(ebuf.at[sl], den_sh.at[srcs], semS, add=True)
            if want_e:
                pltpu.async_copy(ebuf, eO.at[pl.ds(j * _KC, _KC)], semE)

        def drain_scatters(ebuf, rowbuf, semS, semE):
            for off, sz in _HALVES:
                sl = pl.ds(off, sz)
                pltpu.make_async_copy(hT.at[pl.ds(0, sz)], rowbuf.at[sl],
                                      semS).wait()
                pltpu.make_async_copy(ssT.at[pl.ds(0, sz)], ebuf.at[sl],
                                      semS).wait()
            if want_e:
                pltpu.make_async_copy(ebuf, eO.at[pl.ds(0, _KC)], semE).wait()

        issue_idx(wid, srcbA, dstbA, semIA)
        drain_idx(srcbA, dstbA, semIA)
        issue_gathers(srcbA, dstbA, ssrcbA, sdstbA, rowbufA, semGA)

        @pl.when(jnp.int32(1) < nch_mine)
        def _():
            issue_idx(wid + _NW, srcbB, dstbB, semIB)

        def piter(k2, carry):
            k0 = 2 * k2
            j0 = wid + k0 * _NW
            j1 = j0 + _NW
            has1 = k0 + 1 < nch_mine

            @pl.when(has1)
            def _():
                @pl.when(k2 > 0)
                def _():
                    drain_scatters(ebufB, rowbufB, semSB, semEB)
                drain_idx(srcbB, dstbB, semIB)
                issue_gathers(srcbB, dstbB, ssrcbB, sdstbB, rowbufB, semGB)

            drain_gathers(ssrcbA, sdstbA, rowbufA, semGA)
            copy_scatter_idx(srcbA, srcs0A, srcs1A)

            @pl.when(k0 + 2 < nch_mine)
            def _():
                issue_idx(j0 + 2 * _NW, srcbA, dstbA, semIA)

            compute(ssrcbA, sdstbA, ebufA, rowbufA)
            issue_scatters(j0, srcs0A, srcs1A, ebufA, rowbufA, semSA, semEA)

            @pl.when(has1)
            def _():
                drain_gathers(ssrcbB, sdstbB, rowbufB, semGB)
                copy_scatter_idx(srcbB, srcs0B, srcs1B)

                @pl.when(k0 + 3 < nch_mine)
                def _():
                    issue_idx(j1 + 2 * _NW, srcbB, dstbB, semIB)

                compute(ssrcbB, sdstbB, ebufB, rowbufB)
                issue_scatters(j1, srcs0B, srcs1B, ebufB, rowbufB, semSB,
                               semEB)

            @pl.when(k0 + 2 < nch_mine)
            def _():
                drain_scatters(ebufA, rowbufA, semSA, semEA)
                drain_idx(srcbA, dstbA, semIA)
                issue_gathers(srcbA, dstbA, ssrcbA, sdstbA, rowbufA, semGA)

            return carry

        lax.fori_loop(0, nch2, piter, 0)
        drain_scatters(ebufA, rowbufA, semSA, semEA)
        drain_scatters(ebufB, rowbufB, semSB, semEB)
        plsc.subcore_barrier()

        pltpu.sync_copy(acc_sh.at[pl.ds(base, _RPT)],
                        accO.at[c, pl.ds(base, _RPT)])
        pltpu.sync_copy(den_sh.at[pl.ds(base, _RPT)],
                        denO.at[c, pl.ds(base, _RPT)])

    return pl.kernel(
        body,
        out_type=out_type,
        mesh=_get_mesh(),
        scratch_types=[
            pltpu.VMEM_SHARED((_NPAD, _NH), jnp.float32),
            pltpu.VMEM_SHARED((_NPAD,), jnp.float32),
        ] + 2 * [
            pltpu.VMEM((_KC,), jnp.int32),
            pltpu.VMEM((_KC,), jnp.int32),
            pltpu.VMEM((_KC,), jnp.float32),
            pltpu.VMEM((_KC,), jnp.float32),
            pltpu.VMEM((_KC,), jnp.float32),
            pltpu.VMEM((_KC, _NH), jnp.float32),
            pltpu.VMEM((_HALVES[0][1],), jnp.int32),
            pltpu.VMEM((_HALVES[1][1],), jnp.int32),
        ] + 8 * [pltpu.SemaphoreType.DMA],
    )


def _alpha_pass(e2, den0, den1, srcH):
    def body(eH, den0H, den1H, srcH_, aO,
             srcbA, ebufA, dbufA, d1bufA, abufA,
             srcbB, ebufB, dbufB, d1bufB, abufB,
             semGA, semGB, semSA, semSB, semIA, semIB):
        c = lax.axis_index("c")
        s = lax.axis_index("s")
        wid = s * _NC + c
        nch_mine = lax.select(wid < _NCH % _NW,
                              jnp.int32(_NCH // _NW + 1), jnp.int32(_NCH // _NW))
        nch2 = (nch_mine + 1) // 2

        def issue_idx(j, srcb, ebuf, semI):
            pltpu.async_copy(srcH_.at[pl.ds(j * _K, _K)], srcb, semI)
            pltpu.async_copy(eH.at[pl.ds(j * _K, _K)], ebuf, semI)

        def drain_idx(srcb, ebuf, semI):
            pltpu.make_async_copy(srcH_.at[pl.ds(0, _K)], srcb, semI).wait()
            pltpu.make_async_copy(eH.at[pl.ds(0, _K)], ebuf, semI).wait()

        def issue_gather(srcb, dbuf, d1buf, sem):
            pltpu.async_copy(den0H.at[srcb], dbuf, sem)
            pltpu.async_copy(den1H.at[srcb], d1buf, sem)

        def drain_gather(dbuf, d1buf, sem):
            pltpu.make_async_copy(den0H.at[pl.ds(0, _K)], dbuf, sem).wait()
            pltpu.make_async_copy(den1H.at[pl.ds(0, _K)], d1buf, sem).wait()

        def compute(ebuf, dbuf, d1buf, abuf):
            def grp(g, carry2):
                sl = pl.ds(g * 16, 16)
                abuf[sl] = ebuf[sl] / (dbuf[sl] + d1buf[sl])
                return carry2

            lax.fori_loop(0, _K // 16, grp, 0)

        def issue_store(j, abuf, sem):
            pltpu.async_copy(abuf, aO.at[pl.ds(j * _K, _K)], sem)

        def drain_store(abuf, sem):
            pltpu.make_async_copy(abuf, aO.at[pl.ds(0, _K)], sem).wait()

        issue_idx(wid, srcbA, ebufA, semIA)
        drain_idx(srcbA, ebufA, semIA)
        issue_gather(srcbA, dbufA, d1bufA, semGA)

        @pl.when(jnp.int32(1) < nch_mine)
        def _():
            issue_idx(wid + _NW, srcbB, ebufB, semIB)

        def piter(k2, carry):
            k0 = 2 * k2
            j0 = wid + k0 * _NW
            j1 = j0 + _NW
            has1 = k0 + 1 < nch_mine

            @pl.when(has1)
            def _():
                @pl.when(k2 > 0)
                def _():
                    drain_store(abufB, semSB)
                drain_idx(srcbB, ebufB, semIB)
                issue_gather(srcbB, dbufB, d1bufB, semGB)

            drain_gather(dbufA, d1bufA, semGA)
            compute(ebufA, dbufA, d1bufA, abufA)
            issue_store(j0, abufA, semSA)

            @pl.when(k0 + 2 < nch_mine)
            def _():
                issue_idx(j0 + 2 * _NW, srcbA, ebufA, semIA)

            @pl.when(has1)
            def _():
                drain_gather(dbufB, d1bufB, semGB)
                compute(ebufB, dbufB, d1bufB, abufB)
                issue_store(j1, abufB, semSB)

                @pl.when(k0 + 3 < nch_mine)
                def _():
                    issue_idx(j1 + 2 * _NW, srcbB, ebufB, semIB)

            @pl.when(k0 + 2 < nch_mine)
            def _():
                drain_store(abufA, semSA)
                drain_idx(srcbA, ebufA, semIA)
                issue_gather(srcbA, dbufA, d1bufA, semGA)

            return carry

        lax.fori_loop(0, nch2, piter, 0)
        drain_store(abufA, semSA)
        drain_store(abufB, semSB)

    return pl.kernel(
        body,
        out_type=jax.ShapeDtypeStruct((_E,), jnp.float32),
        mesh=_get_mesh(),
        scratch_types=2 * [
            pltpu.VMEM((_K,), jnp.int32),
            pltpu.VMEM((_K,), jnp.float32),
            pltpu.VMEM((_K,), jnp.float32),
            pltpu.VMEM((_K,), jnp.float32),
            pltpu.VMEM((_K,), jnp.float32),
        ] + 6 * [pltpu.SemaphoreType.DMA],
    )(e2, den0, den1, srcH)


def kernel(x, edge_index, W1, a1_w, a1_b, W2, a2_w, a2_b, fc_w, fc_b):
    src = edge_index[0]
    dst = edge_index[1]

    h1, ss1, sd1 = _first_stage(x, W1, a1_w, a1_b)
    acc1, den1 = _make_agg(False)(
        h1, ss1.reshape(_NPAD), sd1.reshape(_NPAD), src, dst)
    h2, ss2, sd2 = _mid_stage(acc1, den1, W2, a2_w, a2_b)
    acc2, den2, e2 = _make_agg(True)(
        h2, ss2.reshape(_NPAD), sd2.reshape(_NPAD), src, dst)
    out = _final_stage(acc2, den2, fc_w, fc_b)
    alpha = _alpha_pass(e2, den2[0], den2[1], src)
    return out[:_N], alpha


# speedup vs baseline: 20.5655x; 1.0056x over previous
import functools

import jax
import jax.numpy as jnp
from jax import lax
from jax.experimental import pallas as pl
from jax.experimental.pallas import tpu as pltpu
from jax.experimental.pallas import tpu_sc as plsc

_N = 10000
_E = 320000
_NH = 128
_NCLASS = 64
_K = 128
_KC = 160
_HALVES = ((0, 128), (128, 32))
_KA = 512
_NCHA = _E // _KA
_NCHC = _E // _KC
_NC = 2
_NS = 16
_NW = _NC * _NS
_NPAD = 10240
_RPT = _NPAD // _NS
_R = 1024


def _dense_tail(h, aw_ref, ab_ref, hp_ref, ss_ref, sd_ref):
    aw_v = aw_ref[...]
    s_src = jnp.dot(h, aw_v[0:128], preferred_element_type=jnp.float32) + ab_ref[0]
    s_dst = jnp.dot(h, aw_v[128:256], preferred_element_type=jnp.float32)
    hp_ref[...] = h
    ss_ref[...] = s_src
    sd_ref[...] = s_dst


_NODE_OUT = [
    jax.ShapeDtypeStruct((_NPAD, _NH), jnp.float32),
    jax.ShapeDtypeStruct((_NPAD, 1), jnp.float32),
    jax.ShapeDtypeStruct((_NPAD, 1), jnp.float32),
]


def _node_specs(R):
    return [
        pl.BlockSpec((R, _NH), lambda i: (i, 0)),
        pl.BlockSpec((R, 1), lambda i: (i, 0)),
        pl.BlockSpec((R, 1), lambda i: (i, 0)),
    ]


def _first_stage(x, W, aw, ab, R=1000):
    def body(x_ref, w_ref, aw_ref, ab_ref, hp_ref, ss_ref, sd_ref):
        h = jnp.dot(x_ref[...], w_ref[...], preferred_element_type=jnp.float32)
        _dense_tail(h, aw_ref, ab_ref, hp_ref, ss_ref, sd_ref)

    return pl.pallas_call(
        body,
        grid=(_N // R,),
        in_specs=[
            pl.BlockSpec((R, _NH), lambda i: (i, 0)),
            pl.BlockSpec((_NH, _NH), lambda i: (0, 0)),
            pl.BlockSpec((2 * _NH, 1), lambda i: (0, 0)),
            pl.BlockSpec(memory_space=pltpu.SMEM),
        ],
        out_specs=_node_specs(R),
        out_shape=_NODE_OUT,
    )(x, W, aw, ab)


def _mid_stage(acc, den, W, aw, ab):
    def body(acc_ref, den_ref, w_ref, aw_ref, ab_ref, hp_ref, ss_ref, sd_ref):
        sacc = acc_ref[0] + acc_ref[1]
        den_v = (den_ref[0] + den_ref[1]).reshape(_R, 1)
        den_v = jnp.where(den_v > 0.0, den_v, 1.0)
        h_in = jnp.maximum(sacc / den_v, 0.0)
        h = jnp.dot(h_in, w_ref[...], preferred_element_type=jnp.float32)
        _dense_tail(h, aw_ref, ab_ref, hp_ref, ss_ref, sd_ref)

    return pl.pallas_call(
        body,
        grid=(_NPAD // _R,),
        in_specs=[
            pl.BlockSpec((2, _R, _NH), lambda i: (0, i, 0)),
            pl.BlockSpec((2, _R), lambda i: (0, i)),
            pl.BlockSpec((_NH, _NH), lambda i: (0, 0)),
            pl.BlockSpec((2 * _NH, 1), lambda i: (0, 0)),
            pl.BlockSpec(memory_space=pltpu.SMEM),
        ],
        out_specs=_node_specs(_R),
        out_shape=_NODE_OUT,
    )(acc, den, W, aw, ab)


def _final_stage(acc, den, fc_w, fc_b):
    def body(acc_ref, den_ref, w_ref, b_ref, out_ref):
        sacc = acc_ref[0] + acc_ref[1]
        den_v = (den_ref[0] + den_ref[1]).reshape(_R, 1)
        den_s = jnp.where(den_v > 0.0, den_v, 1.0)
        h = jnp.maximum(sacc / den_s, 0.0)
        logits = jnp.dot(h, w_ref[...], preferred_element_type=jnp.float32) + b_ref[...]
        m = jnp.max(logits, axis=1, keepdims=True)
        lse = m + jnp.log(jnp.sum(jnp.exp(logits - m), axis=1, keepdims=True))
        out_ref[...] = logits - lse

    return pl.pallas_call(
        body,
        grid=(_NPAD // _R,),
        in_specs=[
            pl.BlockSpec((2, _R, _NH), lambda i: (0, i, 0)),
            pl.BlockSpec((2, _R), lambda i: (0, i)),
            pl.BlockSpec((_NH, _NCLASS), lambda i: (0, 0)),
            pl.BlockSpec((1, _NCLASS), lambda i: (0, 0)),
        ],
        out_specs=pl.BlockSpec((_R, _NCLASS), lambda i: (i, 0)),
        out_shape=jax.ShapeDtypeStruct((_NPAD, _NCLASS), jnp.float32),
    )(acc, den, fc_w, fc_b.reshape(1, _NCLASS))


@functools.cache
def _get_mesh():
    return plsc.VectorSubcoreMesh(core_axis_name="c", subcore_axis_name="s",
                                  num_cores=_NC, num_subcores=_NS)


@functools.cache
def _make_agg(want_e):
    out_type = [
        jax.ShapeDtypeStruct((_NC, _NPAD, _NH), jnp.float32),
        jax.ShapeDtypeStruct((_NC, _NPAD), jnp.float32),
    ]
    if want_e:
        out_type.append(jax.ShapeDtypeStruct((_E,), jnp.float32))

    def body(hT, ssT, sdT, srcH, dstH, *rest):
        if want_e:
            (accO, denO, eO, acc_sh, den_sh,
             srcbA, dstbA, ssrcbA, sdstbA, ebufA, rowbufA, srcs0A, srcs1A,
             srcbB, dstbB, ssrcbB, sdstbB, ebufB, rowbufB, srcs0B, srcs1B,
             semGA, semGB, semSA, semSB, semEA, semEB, semIA, semIB) = rest
        else:
            (accO, denO, acc_sh, den_sh,
             srcbA, dstbA, ssrcbA, sdstbA, ebufA, rowbufA, srcs0A, srcs1A,
             srcbB, dstbB, ssrcbB, sdstbB, ebufB, rowbufB, srcs0B, srcs1B,
             semGA, semGB, semSA, semSB, semEA, semEB, semIA, semIB) = rest
            eO = None
        c = lax.axis_index("c")
        s = lax.axis_index("s")
        wid = s * _NC + c

        zv = jnp.zeros((16,), jnp.float32)

        def zrow(i, carry):
            for g in range(_NH // 16):
                rowbufA[i, pl.ds(g * 16, 16)] = zv
            return carry

        lax.fori_loop(0, _KC, zrow, 0)

        def zden(t, carry):
            ebufA[pl.ds(t * 16, 16)] = zv
            return carry

        lax.fori_loop(0, _KC // 16, zden, 0)

        base = s * _RPT
        for t in range(_RPT // _KC):
            pltpu.sync_copy(rowbufA, acc_sh.at[pl.ds(base + t * _KC, _KC)])
            pltpu.sync_copy(ebufA, den_sh.at[pl.ds(base + t * _KC, _KC)])
        rem = _RPT % _KC
        if rem:
            off = base + (_RPT // _KC) * _KC
            pltpu.sync_copy(rowbufA.at[pl.ds(0, rem)],
                            acc_sh.at[pl.ds(off, rem)])
            pltpu.sync_copy(ebufA.at[pl.ds(0, rem)],
                            den_sh.at[pl.ds(off, rem)])
        plsc.subcore_barrier()

        nch_mine = lax.select(wid < _NCHC % _NW,
                              jnp.int32(_NCHC // _NW + 1),
                              jnp.int32(_NCHC // _NW))
        nch2 = (nch_mine + 1) // 2

        def issue_idx(j, srcb, dstb, semI):
            pltpu.async_copy(srcH.at[pl.ds(j * _KC, _KC)], srcb, semI)
            pltpu.async_copy(dstH.at[pl.ds(j * _KC, _KC)], dstb, semI)

        def drain_idx(srcb, dstb, semI):
            pltpu.make_async_copy(srcH.at[pl.ds(0, _KC)], srcb, semI).wait()
            pltpu.make_async_copy(dstH.at[pl.ds(0, _KC)], dstb, semI).wait()

        def copy_scatter_idx(srcb, srcs0, srcs1):
            for g in range(_HALVES[0][1] // 16):
                srcs0[pl.ds(g * 16, 16)] = srcb[pl.ds(g * 16, 16)]
            for g in range(_HALVES[1][1] // 16):
                srcs1[pl.ds(g * 16, 16)] = srcb[pl.ds(_HALVES[1][0] + g * 16,
                                                      16)]

        def issue_gathers(srcb, dstb, ssrcb, sdstb, rowbuf, sem):
            for off, sz in _HALVES:
                sl = pl.ds(off, sz)
                pltpu.async_copy(ssT.at[srcb.at[sl]], ssrcb.at[sl], sem)
                pltpu.async_copy(sdT.at[dstb.at[sl]], sdstb.at[sl], sem)
                pltpu.async_copy(hT.at[dstb.at[sl]], rowbuf.at[sl], sem)

        def drain_gathers(ssrcb, sdstb, rowbuf, sem):
            for off, sz in _HALVES:
                sl = pl.ds(off, sz)
                pltpu.make_async_copy(ssT.at[pl.ds(0, sz)], ssrcb.at[sl],
                                      sem).wait()
                pltpu.make_async_copy(sdT.at[pl.ds(0, sz)], sdstb.at[sl],
                                      sem).wait()
                pltpu.make_async_copy(hT.at[pl.ds(0, sz)], rowbuf.at[sl],
                                      sem).wait()

        def compute(ssrcb, sdstb, ebuf, rowbuf):
            def egrp(g, carry2):
                sl = pl.ds(g * 16, 16)
                z = ssrcb[sl] + sdstb[sl]
                z = jnp.maximum(z, 0.05 * z)
                ebuf[sl] = jnp.exp(z)
                return carry2

            lax.fori_loop(0, _KC // 16, egrp, 0)

            def rscale16(g, carry2):
                ev16 = ebuf[pl.ds(g * 16, 16)]
                for l in range(16):
                    i = g * 16 + l
                    ev = ev16[l]
                    for q in range(_NH // 16):
                        rowbuf[i, pl.ds(q * 16, 16)] = (
                            rowbuf[i, pl.ds(q * 16, 16)] * ev)
                return carry2

            lax.fori_loop(0, _KC // 16, rscale16, 0)

        def issue_scatters(j, srcs0, srcs1, ebuf, rowbuf, semS, semE):
            for (off, sz), srcs in zip(_HALVES, (srcs0, srcs1)):
                sl = pl.ds(off, sz)
                pltpu.async_copy(rowbuf.at[sl], acc_sh.at[srcs], semS,
                                 add=True)
                pltpu.async_copy(ebuf.at[sl], den_sh.at[srcs], semS, add=True)
            if want_e:
                pltpu.async_copy(ebuf, eO.at[pl.ds(j * _KC, _KC)], semE)

        def drain_scatters(ebuf, rowbuf, semS, semE):
            for off, sz in _HALVES:
                sl = pl.ds(off, sz)
                pltpu.make_async_copy(hT.at[pl.ds(0, sz)], rowbuf.at[sl],
                                      semS).wait()
                pltpu.make_async_copy(ssT.at[pl.ds(0, sz)], ebuf.at[sl],
                                      semS).wait()
            if want_e:
                pltpu.make_async_copy(ebuf, eO.at[pl.ds(0, _KC)], semE).wait()

        issue_idx(wid, srcbA, dstbA, semIA)
        drain_idx(srcbA, dstbA, semIA)
        issue_gathers(srcbA, dstbA, ssrcbA, sdstbA, rowbufA, semGA)

        @pl.when(jnp.int32(1) < nch_mine)
        def _():
            issue_idx(wid + _NW, srcbB, dstbB, semIB)

        def piter(k2, carry):
            k0 = 2 * k2
            j0 = wid + k0 * _NW
            j1 = j0 + _NW
            has1 = k0 + 1 < nch_mine

            @pl.when(has1)
            def _():
                @pl.when(k2 > 0)
                def _():
                    drain_scatters(ebufB, rowbufB, semSB, semEB)
                drain_idx(srcbB, dstbB, semIB)
                issue_gathers(srcbB, dstbB, ssrcbB, sdstbB, rowbufB, semGB)

            drain_gathers(ssrcbA, sdstbA, rowbufA, semGA)
            copy_scatter_idx(srcbA, srcs0A, srcs1A)

            @pl.when(k0 + 2 < nch_mine)
            def _():
                issue_idx(j0 + 2 * _NW, srcbA, dstbA, semIA)

            compute(ssrcbA, sdstbA, ebufA, rowbufA)
            issue_scatters(j0, srcs0A, srcs1A, ebufA, rowbufA, semSA, semEA)

            @pl.when(has1)
            def _():
                drain_gathers(ssrcbB, sdstbB, rowbufB, semGB)
                copy_scatter_idx(srcbB, srcs0B, srcs1B)

                @pl.when(k0 + 3 < nch_mine)
                def _():
                    issue_idx(j1 + 2 * _NW, srcbB, dstbB, semIB)

                compute(ssrcbB, sdstbB, ebufB, rowbufB)
                issue_scatters(j1, srcs0B, srcs1B, ebufB, rowbufB, semSB,
                               semEB)

            @pl.when(k0 + 2 < nch_mine)
            def _():
                drain_scatters(ebufA, rowbufA, semSA, semEA)
                drain_idx(srcbA, dstbA, semIA)
                issue_gathers(srcbA, dstbA, ssrcbA, sdstbA, rowbufA, semGA)

            return carry

        lax.fori_loop(0, nch2, piter, 0)
        drain_scatters(ebufA, rowbufA, semSA, semEA)
        drain_scatters(ebufB, rowbufB, semSB, semEB)
        plsc.subcore_barrier()

        pltpu.sync_copy(acc_sh.at[pl.ds(base, _RPT)],
                        accO.at[c, pl.ds(base, _RPT)])
        pltpu.sync_copy(den_sh.at[pl.ds(base, _RPT)],
                        denO.at[c, pl.ds(base, _RPT)])

    return pl.kernel(
        body,
        out_type=out_type,
        mesh=_get_mesh(),
        scratch_types=[
            pltpu.VMEM_SHARED((_NPAD, _NH), jnp.float32),
            pltpu.VMEM_SHARED((_NPAD,), jnp.float32),
        ] + 2 * [
            pltpu.VMEM((_KC,), jnp.int32),
            pltpu.VMEM((_KC,), jnp.int32),
            pltpu.VMEM((_KC,), jnp.float32),
            pltpu.VMEM((_KC,), jnp.float32),
            pltpu.VMEM((_KC,), jnp.float32),
            pltpu.VMEM((_KC, _NH), jnp.float32),
            pltpu.VMEM((_HALVES[0][1],), jnp.int32),
            pltpu.VMEM((_HALVES[1][1],), jnp.int32),
        ] + 8 * [pltpu.SemaphoreType.DMA],
    )


def _alpha_pass(e2, den0, den1, srcH):
    def body(eH, den0H, den1H, srcH_, aO,
             srcbA, ebufA, dbufA, d1bufA, abufA,
             srcbB, ebufB, dbufB, d1bufB, abufB,
             semGA, semGB, semSA, semSB, semIA, semIB):
        c = lax.axis_index("c")
        s = lax.axis_index("s")
        wid = s * _NC + c
        nch_mine = lax.select(wid < _NCHA % _NW,
                              jnp.int32(_NCHA // _NW + 1),
                              jnp.int32(_NCHA // _NW))
        nch2 = (nch_mine + 1) // 2

        def issue_idx(j, srcb, ebuf, semI):
            pltpu.async_copy(srcH_.at[pl.ds(j * _KA, _KA)], srcb, semI)
            pltpu.async_copy(eH.at[pl.ds(j * _KA, _KA)], ebuf, semI)

        def drain_idx(srcb, ebuf, semI):
            pltpu.make_async_copy(srcH_.at[pl.ds(0, _KA)], srcb, semI).wait()
            pltpu.make_async_copy(eH.at[pl.ds(0, _KA)], ebuf, semI).wait()

        def issue_gather(srcb, dbuf, d1buf, sem):
            for h in range(_KA // _K):
                sl = pl.ds(h * _K, _K)
                pltpu.async_copy(den0H.at[srcb.at[sl]], dbuf.at[sl], sem)
                pltpu.async_copy(den1H.at[srcb.at[sl]], d1buf.at[sl], sem)

        def drain_gather(dbuf, d1buf, sem):
            for h in range(_KA // _K):
                sl = pl.ds(h * _K, _K)
                pltpu.make_async_copy(den0H.at[pl.ds(0, _K)], dbuf.at[sl],
                                      sem).wait()
                pltpu.make_async_copy(den1H.at[pl.ds(0, _K)], d1buf.at[sl],
                                      sem).wait()

        def compute(ebuf, dbuf, d1buf, abuf):
            def grp(g, carry2):
                sl = pl.ds(g * 16, 16)
                abuf[sl] = ebuf[sl] / (dbuf[sl] + d1buf[sl])
                return carry2

            lax.fori_loop(0, _KA // 16, grp, 0)

        def issue_store(j, abuf, sem):
            pltpu.async_copy(abuf, aO.at[pl.ds(j * _KA, _KA)], sem)

        def drain_store(abuf, sem):
            pltpu.make_async_copy(abuf, aO.at[pl.ds(0, _KA)], sem).wait()

        issue_idx(wid, srcbA, ebufA, semIA)
        drain_idx(srcbA, ebufA, semIA)
        issue_gather(srcbA, dbufA, d1bufA, semGA)

        @pl.when(jnp.int32(1) < nch_mine)
        def _():
            issue_idx(wid + _NW, srcbB, ebufB, semIB)

        def piter(k2, carry):
            k0 = 2 * k2
            j0 = wid + k0 * _NW
            j1 = j0 + _NW
            has1 = k0 + 1 < nch_mine

            @pl.when(has1)
            def _():
                @pl.when(k2 > 0)
                def _():
                    drain_store(abufB, semSB)
                drain_idx(srcbB, ebufB, semIB)
                issue_gather(srcbB, dbufB, d1bufB, semGB)

            drain_gather(dbufA, d1bufA, semGA)
            compute(ebufA, dbufA, d1bufA, abufA)
            issue_store(j0, abufA, semSA)

            @pl.when(k0 + 2 < nch_mine)
            def _():
                issue_idx(j0 + 2 * _NW, srcbA, ebufA, semIA)

            @pl.when(has1)
            def _():
                drain_gather(dbufB, d1bufB, semGB)
                compute(ebufB, dbufB, d1bufB, abufB)
                issue_store(j1, abufB, semSB)

                @pl.when(k0 + 3 < nch_mine)
                def _():
                    issue_idx(j1 + 2 * _NW, srcbB, ebufB, semIB)

            @pl.when(k0 + 2 < nch_mine)
            def _():
                drain_store(abufA, semSA)
                drain_idx(srcbA, ebufA, semIA)
                issue_gather(srcbA, dbufA, d1bufA, semGA)

            return carry

        lax.fori_loop(0, nch2, piter, 0)
        drain_store(abufA, semSA)
        drain_store(abufB, semSB)

    return pl.kernel(
        body,
        out_type=jax.ShapeDtypeStruct((_E,), jnp.float32),
        mesh=_get_mesh(),
        scratch_types=2 * [
            pltpu.VMEM((_KA,), jnp.int32),
            pltpu.VMEM((_KA,), jnp.float32),
            pltpu.VMEM((_KA,), jnp.float32),
            pltpu.VMEM((_KA,), jnp.float32),
            pltpu.VMEM((_KA,), jnp.float32),
        ] + 6 * [pltpu.SemaphoreType.DMA],
    )(e2, den0, den1, srcH)


def kernel(x, edge_index, W1, a1_w, a1_b, W2, a2_w, a2_b, fc_w, fc_b):
    src = edge_index[0]
    dst = edge_index[1]

    h1, ss1, sd1 = _first_stage(x, W1, a1_w, a1_b)
    acc1, den1 = _make_agg(False)(
        h1, ss1.reshape(_NPAD), sd1.reshape(_NPAD), src, dst)
    h2, ss2, sd2 = _mid_stage(acc1, den1, W2, a2_w, a2_b)
    acc2, den2, e2 = _make_agg(True)(
        h2, ss2.reshape(_NPAD), sd2.reshape(_NPAD), src, dst)
    out = _final_stage(acc2, den2, fc_w, fc_b)
    alpha = _alpha_pass(e2, den2[0], den2[1], src)
    return out[:_N], alpha


# speedup vs baseline: 20.6146x; 1.0024x over previous
import functools

import jax
import jax.numpy as jnp
from jax import lax
from jax.experimental import pallas as pl
from jax.experimental.pallas import tpu as pltpu
from jax.experimental.pallas import tpu_sc as plsc

_N = 10000
_E = 320000
_NH = 128
_NCLASS = 64
_K = 128
_KC = 160
_HALVES = ((0, 128), (128, 32))
_KA = 512
_NCHA = _E // _KA
_NCHC = _E // _KC
_NC = 2
_NS = 16
_NW = _NC * _NS
_NPAD = 10240
_RPT = _NPAD // _NS
_R = 1024


def _dense_tail(h, aw_ref, ab_ref, hp_ref, ss_ref, sd_ref):
    aw_v = aw_ref[...]
    s_src = jnp.dot(h, aw_v[0:128], preferred_element_type=jnp.float32) + ab_ref[0]
    s_dst = jnp.dot(h, aw_v[128:256], preferred_element_type=jnp.float32)
    hp_ref[...] = h
    ss_ref[...] = s_src
    sd_ref[...] = s_dst


_NODE_OUT = [
    jax.ShapeDtypeStruct((_NPAD, _NH), jnp.float32),
    jax.ShapeDtypeStruct((_NPAD, 1), jnp.float32),
    jax.ShapeDtypeStruct((_NPAD, 1), jnp.float32),
]


def _node_specs(R):
    return [
        pl.BlockSpec((R, _NH), lambda i: (i, 0)),
        pl.BlockSpec((R, 1), lambda i: (i, 0)),
        pl.BlockSpec((R, 1), lambda i: (i, 0)),
    ]


def _first_stage(x, W, aw, ab, R=1000):
    def body(x_ref, w_ref, aw_ref, ab_ref, hp_ref, ss_ref, sd_ref):
        h = jnp.dot(x_ref[...], w_ref[...], preferred_element_type=jnp.float32)
        _dense_tail(h, aw_ref, ab_ref, hp_ref, ss_ref, sd_ref)

    return pl.pallas_call(
        body,
        grid=(_N // R,),
        in_specs=[
            pl.BlockSpec((R, _NH), lambda i: (i, 0)),
            pl.BlockSpec((_NH, _NH), lambda i: (0, 0)),
            pl.BlockSpec((2 * _NH, 1), lambda i: (0, 0)),
            pl.BlockSpec(memory_space=pltpu.SMEM),
        ],
        out_specs=_node_specs(R),
        out_shape=_NODE_OUT,
    )(x, W, aw, ab)


def _mid_stage(acc, den, W, aw, ab):
    def body(acc_ref, den_ref, w_ref, aw_ref, ab_ref, hp_ref, ss_ref, sd_ref):
        sacc = acc_ref[0] + acc_ref[1]
        den_v = (den_ref[0] + den_ref[1]).reshape(_R, 1)
        den_v = jnp.where(den_v > 0.0, den_v, 1.0)
        h_in = jnp.maximum(sacc / den_v, 0.0)
        h = jnp.dot(h_in, w_ref[...], preferred_element_type=jnp.float32)
        _dense_tail(h, aw_ref, ab_ref, hp_ref, ss_ref, sd_ref)

    return pl.pallas_call(
        body,
        grid=(_NPAD // _R,),
        in_specs=[
            pl.BlockSpec((2, _R, _NH), lambda i: (0, i, 0)),
            pl.BlockSpec((2, _R), lambda i: (0, i)),
            pl.BlockSpec((_NH, _NH), lambda i: (0, 0)),
            pl.BlockSpec((2 * _NH, 1), lambda i: (0, 0)),
            pl.BlockSpec(memory_space=pltpu.SMEM),
        ],
        out_specs=_node_specs(_R),
        out_shape=_NODE_OUT,
    )(acc, den, W, aw, ab)


def _final_stage(acc, den0, den1, fc_w, fc_b, R=1000):
    def body(acc_ref, den0_ref, den1_ref, w_ref, b_ref, out_ref):
        sacc = acc_ref[0] + acc_ref[1]
        den_v = den0_ref[...] + den1_ref[...]
        den_s = jnp.where(den_v > 0.0, den_v, 1.0)
        h = jnp.maximum(sacc / den_s, 0.0)
        logits = jnp.dot(h, w_ref[...], preferred_element_type=jnp.float32) + b_ref[...]
        m = jnp.max(logits, axis=1, keepdims=True)
        lse = m + jnp.log(jnp.sum(jnp.exp(logits - m), axis=1, keepdims=True))
        out_ref[...] = logits - lse

    return pl.pallas_call(
        body,
        grid=(_N // R,),
        in_specs=[
            pl.BlockSpec((2, R, _NH), lambda i: (0, i, 0)),
            pl.BlockSpec((R, 1), lambda i: (i, 0)),
            pl.BlockSpec((R, 1), lambda i: (i, 0)),
            pl.BlockSpec((_NH, _NCLASS), lambda i: (0, 0)),
            pl.BlockSpec((1, _NCLASS), lambda i: (0, 0)),
        ],
        out_specs=pl.BlockSpec((R, _NCLASS), lambda i: (i, 0)),
        out_shape=jax.ShapeDtypeStruct((_N, _NCLASS), jnp.float32),
    )(acc, den0, den1, fc_w, fc_b.reshape(1, _NCLASS))


@functools.cache
def _get_mesh():
    return plsc.VectorSubcoreMesh(core_axis_name="c", subcore_axis_name="s",
                                  num_cores=_NC, num_subcores=_NS)


@functools.cache
def _make_agg(want_e):
    out_type = [
        jax.ShapeDtypeStruct((_NC, _NPAD, _NH), jnp.float32),
        jax.ShapeDtypeStruct((_NC, _NPAD), jnp.float32),
    ]
    if want_e:
        out_type.append(jax.ShapeDtypeStruct((_E,), jnp.float32))

    def body(hT, ssT, sdT, srcH, dstH, *rest):
        if want_e:
            (accO, denO, eO, acc_sh, den_sh,
             srcbA, dstbA, ssrcbA, sdstbA, ebufA, rowbufA, srcs0A, srcs1A,
             srcbB, dstbB, ssrcbB, sdstbB, ebufB, rowbufB, srcs0B, srcs1B,
             semGA, semGB, semSA, semSB, semEA, semEB, semIA, semIB) = rest
        else:
            (accO, denO, acc_sh, den_sh,
             srcbA, dstbA, ssrcbA, sdstbA, ebufA, rowbufA, srcs0A, srcs1A,
             srcbB, dstbB, ssrcbB, sdstbB, ebufB, rowbufB, srcs0B, srcs1B,
             semGA, semGB, semSA, semSB, semEA, semEB, semIA, semIB) = rest
            eO = None
        c = lax.axis_index("c")
        s = lax.axis_index("s")
        wid = s * _NC + c

        zv = jnp.zeros((16,), jnp.float32)

        def zrow(i, carry):
            for g in range(_NH // 16):
                rowbufA[i, pl.ds(g * 16, 16)] = zv
            return carry

        lax.fori_loop(0, _KC, zrow, 0)

        def zden(t, carry):
            ebufA[pl.ds(t * 16, 16)] = zv
            return carry

        lax.fori_loop(0, _KC // 16, zden, 0)

        base = s * _RPT
        for t in range(_RPT // _KC):
            pltpu.sync_copy(rowbufA, acc_sh.at[pl.ds(base + t * _KC, _KC)])
            pltpu.sync_copy(ebufA, den_sh.at[pl.ds(base + t * _KC, _KC)])
        rem = _RPT % _KC
        if rem:
            off = base + (_RPT // _KC) * _KC
            pltpu.sync_copy(rowbufA.at[pl.ds(0, rem)],
                            acc_sh.at[pl.ds(off, rem)])
            pltpu.sync_copy(ebufA.at[pl.ds(0, rem)],
                            den_sh.at[pl.ds(off, rem)])
        plsc.subcore_barrier()

        nch_mine = lax.select(wid < _NCHC % _NW,
                              jnp.int32(_NCHC // _NW + 1),
                              jnp.int32(_NCHC // _NW))
        nch2 = (nch_mine + 1) // 2

        def issue_idx(j, srcb, dstb, semI):
            pltpu.async_copy(srcH.at[pl.ds(j * _KC, _KC)], srcb, semI)
            pltpu.async_copy(dstH.at[pl.ds(j * _KC, _KC)], dstb, semI)

        def drain_idx(srcb, dstb, semI):
            pltpu.make_async_copy(srcH.at[pl.ds(0, _KC)], srcb, semI).wait()
            pltpu.make_async_copy(dstH.at[pl.ds(0, _KC)], dstb, semI).wait()

        def copy_scatter_idx(srcb, srcs0, srcs1):
            for g in range(_HALVES[0][1] // 16):
                srcs0[pl.ds(g * 16, 16)] = srcb[pl.ds(g * 16, 16)]
            for g in range(_HALVES[1][1] // 16):
                srcs1[pl.ds(g * 16, 16)] = srcb[pl.ds(_HALVES[1][0] + g * 16,
                                                      16)]

        def issue_gathers(srcb, dstb, ssrcb, sdstb, rowbuf, sem):
            for off, sz in _HALVES:
                sl = pl.ds(off, sz)
                pltpu.async_copy(ssT.at[srcb.at[sl]], ssrcb.at[sl], sem)
                pltpu.async_copy(sdT.at[dstb.at[sl]], sdstb.at[sl], sem)
                pltpu.async_copy(hT.at[dstb.at[sl]], rowbuf.at[sl], sem)

        def drain_gathers(ssrcb, sdstb, rowbuf, sem):
            for off, sz in _HALVES:
                sl = pl.ds(off, sz)
                pltpu.make_async_copy(ssT.at[pl.ds(0, sz)], ssrcb.at[sl],
                                      sem).wait()
                pltpu.make_async_copy(sdT.at[pl.ds(0, sz)], sdstb.at[sl],
                                      sem).wait()
                pltpu.make_async_copy(hT.at[pl.ds(0, sz)], rowbuf.at[sl],
                                      sem).wait()

        def compute(ssrcb, sdstb, ebuf, rowbuf):
            def egrp(g, carry2):
                sl = pl.ds(g * 16, 16)
                z = ssrcb[sl] + sdstb[sl]
                z = jnp.maximum(z, 0.05 * z)
                ebuf[sl] = jnp.exp(z)
                return carry2

            lax.fori_loop(0, _KC // 16, egrp, 0)

            def rscale16(g, carry2):
                ev16 = ebuf[pl.ds(g * 16, 16)]
                for l in range(16):
                    i = g * 16 + l
                    ev = ev16[l]
                    for q in range(_NH // 16):
                        rowbuf[i, pl.ds(q * 16, 16)] = (
                            rowbuf[i, pl.ds(q * 16, 16)] * ev)
                return carry2

            lax.fori_loop(0, _KC // 16, rscale16, 0)

        def issue_scatters(j, srcs0, srcs1, ebuf, rowbuf, semS, semE):
            for (off, sz), srcs in zip(_HALVES, (srcs0, srcs1)):
                sl = pl.ds(off, sz)
                pltpu.async_copy(rowbuf.at[sl], acc_sh.at[srcs], semS,
                                 add=True)
                pltpu.async_copy(ebuf.at[sl], den_sh.at[srcs], semS, add=True)
            if want_e:
                pltpu.async_copy(ebuf, eO.at[pl.ds(j * _KC, _KC)], semE)

        def drain_scatters(ebuf, rowbuf, semS, semE):
            for off, sz in _HALVES:
                sl = pl.ds(off, sz)
                pltpu.make_async_copy(hT.at[pl.ds(0, sz)], rowbuf.at[sl],
                                      semS).wait()
                pltpu.make_async_copy(ssT.at[pl.ds(0, sz)], ebuf.at[sl],
                                      semS).wait()
            if want_e:
                pltpu.make_async_copy(ebuf, eO.at[pl.ds(0, _KC)], semE).wait()

        issue_idx(wid, srcbA, dstbA, semIA)
        drain_idx(srcbA, dstbA, semIA)
        issue_gathers(srcbA, dstbA, ssrcbA, sdstbA, rowbufA, semGA)

        @pl.when(jnp.int32(1) < nch_mine)
        def _():
            issue_idx(wid + _NW, srcbB, dstbB, semIB)

        def piter(k2, carry):
            k0 = 2 * k2
            j0 = wid + k0 * _NW
            j1 = j0 + _NW
            has1 = k0 + 1 < nch_mine

            @pl.when(has1)
            def _():
                @pl.when(k2 > 0)
                def _():
                    drain_scatters(ebufB, rowbufB, semSB, semEB)
                drain_idx(srcbB, dstbB, semIB)
                issue_gathers(srcbB, dstbB, ssrcbB, sdstbB, rowbufB, semGB)

            drain_gathers(ssrcbA, sdstbA, rowbufA, semGA)
            copy_scatter_idx(srcbA, srcs0A, srcs1A)

            @pl.when(k0 + 2 < nch_mine)
            def _():
                issue_idx(j0 + 2 * _NW, srcbA, dstbA, semIA)

            compute(ssrcbA, sdstbA, ebufA, rowbufA)
            issue_scatters(j0, srcs0A, srcs1A, ebufA, rowbufA, semSA, semEA)

            @pl.when(has1)
            def _():
                drain_gathers(ssrcbB, sdstbB, rowbufB, semGB)
                copy_scatter_idx(srcbB, srcs0B, srcs1B)

                @pl.when(k0 + 3 < nch_mine)
                def _():
                    issue_idx(j1 + 2 * _NW, srcbB, dstbB, semIB)

                compute(ssrcbB, sdstbB, ebufB, rowbufB)
                issue_scatters(j1, srcs0B, srcs1B, ebufB, rowbufB, semSB,
                               semEB)

            @pl.when(k0 + 2 < nch_mine)
            def _():
                drain_scatters(ebufA, rowbufA, semSA, semEA)
                drain_idx(srcbA, dstbA, semIA)
                issue_gathers(srcbA, dstbA, ssrcbA, sdstbA, rowbufA, semGA)

            return carry

        lax.fori_loop(0, nch2, piter, 0)
        drain_scatters(ebufA, rowbufA, semSA, semEA)
        drain_scatters(ebufB, rowbufB, semSB, semEB)
        plsc.subcore_barrier()

        pltpu.sync_copy(acc_sh.at[pl.ds(base, _RPT)],
                        accO.at[c, pl.ds(base, _RPT)])
        pltpu.sync_copy(den_sh.at[pl.ds(base, _RPT)],
                        denO.at[c, pl.ds(base, _RPT)])

    return pl.kernel(
        body,
        out_type=out_type,
        mesh=_get_mesh(),
        scratch_types=[
            pltpu.VMEM_SHARED((_NPAD, _NH), jnp.float32),
            pltpu.VMEM_SHARED((_NPAD,), jnp.float32),
        ] + 2 * [
            pltpu.VMEM((_KC,), jnp.int32),
            pltpu.VMEM((_KC,), jnp.int32),
            pltpu.VMEM((_KC,), jnp.float32),
            pltpu.VMEM((_KC,), jnp.float32),
            pltpu.VMEM((_KC,), jnp.float32),
            pltpu.VMEM((_KC, _NH), jnp.float32),
            pltpu.VMEM((_HALVES[0][1],), jnp.int32),
            pltpu.VMEM((_HALVES[1][1],), jnp.int32),
        ] + 8 * [pltpu.SemaphoreType.DMA],
    )


def _alpha_pass(e2, den0, den1, srcH):
    def body(eH, den0H, den1H, srcH_, aO,
             srcbA, ebufA, dbufA, d1bufA, abufA,
             srcbB, ebufB, dbufB, d1bufB, abufB,
             semGA, semGB, semSA, semSB, semIA, semIB):
        c = lax.axis_index("c")
        s = lax.axis_index("s")
        wid = s * _NC + c
        nch_mine = lax.select(wid < _NCHA % _NW,
                              jnp.int32(_NCHA // _NW + 1),
                              jnp.int32(_NCHA // _NW))
        nch2 = (nch_mine + 1) // 2

        def issue_idx(j, srcb, ebuf, semI):
            pltpu.async_copy(srcH_.at[pl.ds(j * _KA, _KA)], srcb, semI)
            pltpu.async_copy(eH.at[pl.ds(j * _KA, _KA)], ebuf, semI)

        def drain_idx(srcb, ebuf, semI):
            pltpu.make_async_copy(srcH_.at[pl.ds(0, _KA)], srcb, semI).wait()
            pltpu.make_async_copy(eH.at[pl.ds(0, _KA)], ebuf, semI).wait()

        def issue_gather(srcb, dbuf, d1buf, sem):
            for h in range(_KA // _K):
                sl = pl.ds(h * _K, _K)
                pltpu.async_copy(den0H.at[srcb.at[sl]], dbuf.at[sl], sem)
                pltpu.async_copy(den1H.at[srcb.at[sl]], d1buf.at[sl], sem)

        def drain_gather(dbuf, d1buf, sem):
            for h in range(_KA // _K):
                sl = pl.ds(h * _K, _K)
                pltpu.make_async_copy(den0H.at[pl.ds(0, _K)], dbuf.at[sl],
                                      sem).wait()
                pltpu.make_async_copy(den1H.at[pl.ds(0, _K)], d1buf.at[sl],
                                      sem).wait()

        def compute(ebuf, dbuf, d1buf, abuf):
            def grp(g, carry2):
                sl = pl.ds(g * 16, 16)
                abuf[sl] = ebuf[sl] / (dbuf[sl] + d1buf[sl])
                return carry2

            lax.fori_loop(0, _KA // 16, grp, 0)

        def issue_store(j, abuf, sem):
            pltpu.async_copy(abuf, aO.at[pl.ds(j * _KA, _KA)], sem)

        def drain_store(abuf, sem):
            pltpu.make_async_copy(abuf, aO.at[pl.ds(0, _KA)], sem).wait()

        issue_idx(wid, srcbA, ebufA, semIA)
        drain_idx(srcbA, ebufA, semIA)
        issue_gather(srcbA, dbufA, d1bufA, semGA)

        @pl.when(jnp.int32(1) < nch_mine)
        def _():
            issue_idx(wid + _NW, srcbB, ebufB, semIB)

        def piter(k2, carry):
            k0 = 2 * k2
            j0 = wid + k0 * _NW
            j1 = j0 + _NW
            has1 = k0 + 1 < nch_mine

            @pl.when(has1)
            def _():
                @pl.when(k2 > 0)
                def _():
                    drain_store(abufB, semSB)
                drain_idx(srcbB, ebufB, semIB)
                issue_gather(srcbB, dbufB, d1bufB, semGB)

            drain_gather(dbufA, d1bufA, semGA)
            compute(ebufA, dbufA, d1bufA, abufA)
            issue_store(j0, abufA, semSA)

            @pl.when(k0 + 2 < nch_mine)
            def _():
                issue_idx(j0 + 2 * _NW, srcbA, ebufA, semIA)

            @pl.when(has1)
            def _():
                drain_gather(dbufB, d1bufB, semGB)
                compute(ebufB, dbufB, d1bufB, abufB)
                issue_store(j1, abufB, semSB)

                @pl.when(k0 + 3 < nch_mine)
                def _():
                    issue_idx(j1 + 2 * _NW, srcbB, ebufB, semIB)

            @pl.when(k0 + 2 < nch_mine)
            def _():
                drain_store(abufA, semSA)
                drain_idx(srcbA, ebufA, semIA)
                issue_gather(srcbA, dbufA, d1bufA, semGA)

            return carry

        lax.fori_loop(0, nch2, piter, 0)
        drain_store(abufA, semSA)
        drain_store(abufB, semSB)

    return pl.kernel(
        body,
        out_type=jax.ShapeDtypeStruct((_E,), jnp.float32),
        mesh=_get_mesh(),
        scratch_types=2 * [
            pltpu.VMEM((_KA,), jnp.int32),
            pltpu.VMEM((_KA,), jnp.float32),
            pltpu.VMEM((_KA,), jnp.float32),
            pltpu.VMEM((_KA,), jnp.float32),
            pltpu.VMEM((_KA,), jnp.float32),
        ] + 6 * [pltpu.SemaphoreType.DMA],
    )(e2, den0, den1, srcH)


def kernel(x, edge_index, W1, a1_w, a1_b, W2, a2_w, a2_b, fc_w, fc_b):
    src = edge_index[0]
    dst = edge_index[1]

    h1, ss1, sd1 = _first_stage(x, W1, a1_w, a1_b)
    acc1, den1 = _make_agg(False)(
        h1, ss1.reshape(_NPAD), sd1.reshape(_NPAD), src, dst)
    h2, ss2, sd2 = _mid_stage(acc1, den1, W2, a2_w, a2_b)
    acc2, den2, e2 = _make_agg(True)(
        h2, ss2.reshape(_NPAD), sd2.reshape(_NPAD), src, dst)
    out = _final_stage(acc2, den2[0].reshape(_NPAD, 1),
                       den2[1].reshape(_NPAD, 1), fc_w, fc_b)
    alpha = _alpha_pass(e2, den2[0], den2[1], src)
    return out, alpha
